# Initial kernel scaffold; baseline (speedup 1.0000x reference)
import jax
import jax.numpy as jnp
from jax.experimental import pallas as pl


def kernel(X, W1, b1, W2, b2, node_idx, edge_idx):
    raise NotImplementedError("write your pallas kernel here")


# speedup vs baseline: 3.1246x; 3.1246x over previous
import functools

import jax
import jax.numpy as jnp
from jax import lax
from jax.experimental import pallas as pl
from jax.experimental.pallas import tpu as pltpu
from jax.experimental.pallas import tpu_sc as plsc

N = 10000
NNZ = 160000
D = 256
DH = 128
NS = 16
PER_TILE = NNZ // NS
CHUNK = 80
NCHUNK = PER_TILE // CHUNK
NPAD = 10240
RPT = NPAD // NS
OCH = 80
DEGW = 16
BM = 1000
GRID = N // BM

_f32 = jnp.float32
_mesh = plsc.VectorSubcoreMesh(core_axis_name="c", subcore_axis_name="s")


@functools.partial(
    pl.kernel,
    mesh=_mesh,
    out_type=jax.ShapeDtypeStruct((2 * N, DEGW), _f32),
    scratch_types=[pltpu.VMEM((CHUNK,), jnp.int32),
                   pltpu.VMEM((CHUNK, DEGW), _f32),
                   pltpu.VMEM((OCH, DEGW), _f32),
                   pltpu.VMEM((OCH, DEGW), _f32),
                   pltpu.VMEM_SHARED((NPAD, DEGW), _f32)],
)
def _sc_degrees(cidx, deg_out, idxb, onesb, zb, ob, acc):
    c = lax.axis_index("c")
    s = lax.axis_index("s")
    ones16 = jnp.ones((16,), _f32)
    zero16 = jnp.zeros((16,), _f32)
    for i in range(CHUNK):
        onesb[i, :] = ones16
    for i in range(OCH):
        zb[i, :] = zero16

    def zbody(k, carry):
        pltpu.sync_copy(zb, acc.at[pl.ds(s * RPT + k * OCH, OCH)])
        return carry
    lax.fori_loop(0, RPT // OCH, zbody, None)
    plsc.subcore_barrier()

    def body(j, carry):
        base = c * NNZ + s * PER_TILE + j * CHUNK
        pltpu.sync_copy(cidx.at[pl.ds(base, CHUNK)], idxb)
        pltpu.sync_copy(onesb, acc.at[idxb], add=True)
        return carry
    lax.fori_loop(0, NCHUNK, body, None)
    plsc.subcore_barrier()

    nch = jnp.minimum(jnp.maximum(N - s * RPT, 0), RPT) // OCH

    def obody(k, carry):
        r = s * RPT + k * OCH
        pltpu.sync_copy(acc.at[pl.ds(r, OCH)], ob)
        pltpu.sync_copy(ob, deg_out.at[pl.ds(c * N + r, OCH)])
        return carry
    lax.fori_loop(0, nch, obody, None)


@functools.partial(
    pl.kernel,
    mesh=_mesh,
    out_type=jax.ShapeDtypeStruct((2 * N, DH), _f32),
    scratch_types=[pltpu.VMEM((CHUNK,), jnp.int32),
                   pltpu.VMEM((CHUNK,), jnp.int32),
                   pltpu.VMEM((CHUNK, DH), _f32),
                   pltpu.VMEM((OCH, DH), _f32),
                   pltpu.VMEM((OCH, DH), _f32),
                   pltpu.VMEM_SHARED((NPAD, DH), _f32),
                   pltpu.SemaphoreType.DMA],
)
def _sc_segsum(src_idx, dst_idx, tab, out,
               sidxb, didxb, rows, zb, ob, acc, sem):
    c = lax.axis_index("c")
    s = lax.axis_index("s")
    zero16 = jnp.zeros((16,), _f32)
    for i in range(OCH):
        for k in range(DH // 16):
            zb[i, pl.ds(k * 16, 16)] = zero16

    def zbody(k, carry):
        pltpu.sync_copy(zb, acc.at[pl.ds(s * RPT + k * OCH, OCH)])
        return carry
    lax.fori_loop(0, RPT // OCH, zbody, None)
    plsc.subcore_barrier()

    def body(j, carry):
        base = s * PER_TILE + j * CHUNK
        pltpu.sync_copy(src_idx.at[pl.ds(base, CHUNK)], sidxb)
        pltpu.sync_copy(dst_idx.at[pl.ds(base, CHUNK)], didxb)
        coff = c * N
        for k in range(CHUNK // 16):
            sidxb[pl.ds(k * 16, 16)] = sidxb[pl.ds(k * 16, 16)] + coff
        pltpu.async_copy(tab.at[sidxb], rows, sem).wait()
        pltpu.sync_copy(rows, acc.at[didxb], add=True)
        return carry
    lax.fori_loop(0, NCHUNK, body, None)
    plsc.subcore_barrier()

    nch = jnp.minimum(jnp.maximum(N - s * RPT, 0), RPT) // OCH

    def obody(k, carry):
        r = s * RPT + k * OCH
        pltpu.sync_copy(acc.at[pl.ds(r, OCH)], ob)
        pltpu.sync_copy(ob, out.at[pl.ds(c * N + r, OCH)])
        return carry
    lax.fori_loop(0, nch, obody, None)


def _prep_body(dvd, ded, dv, de):
    dv[...] = lax.rsqrt(dvd[...])
    de[...] = 1.0 / ded[...]


def _tc_prep(deg):
    return pl.pallas_call(
        _prep_body,
        grid=(GRID,),
        in_specs=[pl.BlockSpec((BM, DEGW), lambda i: (i, 0)),
                  pl.BlockSpec((BM, DEGW), lambda i: (GRID + i, 0))],
        out_specs=[pl.BlockSpec((BM, DEGW), lambda i: (i, 0)),
                   pl.BlockSpec((BM, DEGW), lambda i: (i, 0))],
        out_shape=[jax.ShapeDtypeStruct((N, DEGW), _f32),
                   jax.ShapeDtypeStruct((N, DEGW), _f32)],
    )(deg, deg)


def _mm1_body(x, w, b, dv, y):
    yy = lax.dot_general(x[...], w[...], (((1,), (1,)), ((), ())),
                         preferred_element_type=_f32)
    y[...] = (yy + b[...]) * dv[...][:, :1]


def _tc_mm1(x, w1, b1r, dv):
    return pl.pallas_call(
        _mm1_body,
        grid=(GRID, 2),
        in_specs=[pl.BlockSpec((BM, D), lambda i, j: (i, 0)),
                  pl.BlockSpec((DH, D), lambda i, j: (j, 0)),
                  pl.BlockSpec((1, DH), lambda i, j: (0, j)),
                  pl.BlockSpec((BM, DEGW), lambda i, j: (i, 0))],
        out_specs=pl.BlockSpec((BM, DH), lambda i, j: (j * GRID + i, 0)),
        out_shape=jax.ShapeDtypeStruct((2 * N, DH), _f32),
    )(x, w1, b1r, dv)


def _scale_body(z, de, o):
    o[...] = z[...] * de[...][:, :1]


def _tc_scale(z, de):
    return pl.pallas_call(
        _scale_body,
        grid=(GRID, 2),
        in_specs=[pl.BlockSpec((BM, DH), lambda i, j: (j * GRID + i, 0)),
                  pl.BlockSpec((BM, DEGW), lambda i, j: (i, 0))],
        out_specs=pl.BlockSpec((BM, DH), lambda i, j: (j * GRID + i, 0)),
        out_shape=jax.ShapeDtypeStruct((2 * N, DH), _f32),
    )(z, de)


def _mid_body(za, zbr, dv, w, b, y):
    d = dv[...][:, :1]
    h = jnp.concatenate([jnp.maximum(za[...] * d, 0.0),
                         jnp.maximum(zbr[...] * d, 0.0)], axis=1)
    yy = lax.dot_general(h, w[...], (((1,), (1,)), ((), ())),
                         preferred_element_type=_f32)
    y[...] = (yy + b[...]) * d


def _tc_mid(zv, dv, w2, b2r):
    return pl.pallas_call(
        _mid_body,
        grid=(GRID, 2),
        in_specs=[pl.BlockSpec((BM, DH), lambda i, j: (i, 0)),
                  pl.BlockSpec((BM, DH), lambda i, j: (GRID + i, 0)),
                  pl.BlockSpec((BM, DEGW), lambda i, j: (i, 0)),
                  pl.BlockSpec((DH, D), lambda i, j: (j, 0)),
                  pl.BlockSpec((1, DH), lambda i, j: (0, j))],
        out_specs=pl.BlockSpec((BM, DH), lambda i, j: (j * GRID + i, 0)),
        out_shape=jax.ShapeDtypeStruct((2 * N, DH), _f32),
    )(zv, zv, dv, w2, b2r)


def _final_body(za, zbr, dv, o):
    d = dv[...][:, :1]
    o[...] = jnp.concatenate([za[...] * d, zbr[...] * d], axis=1)


def _tc_final(zv, dv):
    return pl.pallas_call(
        _final_body,
        grid=(GRID,),
        in_specs=[pl.BlockSpec((BM, DH), lambda i: (i, 0)),
                  pl.BlockSpec((BM, DH), lambda i: (GRID + i, 0)),
                  pl.BlockSpec((BM, DEGW), lambda i: (i, 0))],
        out_specs=pl.BlockSpec((BM, D), lambda i: (i, 0)),
        out_shape=jax.ShapeDtypeStruct((N, D), _f32),
    )(zv, zv, dv)


def kernel(X, W1, b1, W2, b2, node_idx, edge_idx):
    b1r = b1.reshape(1, D)
    b2r = b2.reshape(1, D)
    cidx = jnp.concatenate([node_idx, edge_idx])
    deg = _sc_degrees(cidx)
    dv, de = _tc_prep(deg)
    y1 = _tc_mm1(X, W1, b1r, dv)
    ze = _sc_segsum(node_idx, edge_idx, y1)
    ze = _tc_scale(ze, de)
    zv = _sc_segsum(edge_idx, node_idx, ze)
    y2 = _tc_mid(zv, dv, W2, b2r)
    z2 = _sc_segsum(node_idx, edge_idx, y2)
    z2 = _tc_scale(z2, de)
    z2 = _sc_segsum(edge_idx, node_idx, z2)
    return _tc_final(z2, dv)


# speedup vs baseline: 3.8258x; 1.2244x over previous
import functools

import jax
import jax.numpy as jnp
from jax import lax
from jax.experimental import pallas as pl
from jax.experimental.pallas import tpu as pltpu
from jax.experimental.pallas import tpu_sc as plsc

N = 10000
NNZ = 160000
D = 256
DH = 128
NS = 16
PER_TILE = NNZ // NS
CHUNK = 80
NCHUNK = PER_TILE // CHUNK
NPAD = 10240
RPT = NPAD // NS
OCH = 80
DEGW = 16
BM = 1000
GRID = N // BM

_f32 = jnp.float32
_mesh = plsc.VectorSubcoreMesh(core_axis_name="c", subcore_axis_name="s")


@functools.partial(
    pl.kernel,
    mesh=_mesh,
    out_type=jax.ShapeDtypeStruct((2 * N, DEGW), _f32),
    scratch_types=[pltpu.VMEM((CHUNK,), jnp.int32),
                   pltpu.VMEM((CHUNK, DEGW), _f32),
                   pltpu.VMEM((OCH, DEGW), _f32),
                   pltpu.VMEM((OCH, DEGW), _f32),
                   pltpu.VMEM_SHARED((NPAD, DEGW), _f32)],
)
def _sc_degrees(cidx, deg_out, idxb, onesb, zb, ob, acc):
    c = lax.axis_index("c")
    s = lax.axis_index("s")
    ones16 = jnp.ones((16,), _f32)
    zero16 = jnp.zeros((16,), _f32)
    for i in range(CHUNK):
        onesb[i, :] = ones16
    for i in range(OCH):
        zb[i, :] = zero16

    def zbody(k, carry):
        pltpu.sync_copy(zb, acc.at[pl.ds(s * RPT + k * OCH, OCH)])
        return carry
    lax.fori_loop(0, RPT // OCH, zbody, None)
    plsc.subcore_barrier()

    def body(j, carry):
        base = c * NNZ + s * PER_TILE + j * CHUNK
        pltpu.sync_copy(cidx.at[pl.ds(base, CHUNK)], idxb)
        pltpu.sync_copy(onesb, acc.at[idxb], add=True)
        return carry
    lax.fori_loop(0, NCHUNK, body, None)
    plsc.subcore_barrier()

    nch = jnp.minimum(jnp.maximum(N - s * RPT, 0), RPT) // OCH

    def obody(k, carry):
        r = s * RPT + k * OCH
        pltpu.sync_copy(acc.at[pl.ds(r, OCH)], ob)
        pltpu.sync_copy(ob, deg_out.at[pl.ds(c * N + r, OCH)])
        return carry
    lax.fori_loop(0, nch, obody, None)


@functools.partial(
    pl.kernel,
    mesh=_mesh,
    out_type=jax.ShapeDtypeStruct((2 * N, DH), _f32),
    scratch_types=[pltpu.VMEM((2 * CHUNK,), jnp.int32),
                   pltpu.VMEM((CHUNK,), jnp.int32),
                   pltpu.VMEM((CHUNK,), jnp.int32),
                   pltpu.VMEM((CHUNK, DH), _f32),
                   pltpu.VMEM((CHUNK, DH), _f32),
                   pltpu.VMEM((OCH, DH), _f32),
                   pltpu.VMEM((OCH, DH), _f32),
                   pltpu.VMEM_SHARED((NPAD, DH), _f32),
                   pltpu.SemaphoreType.DMA,
                   pltpu.SemaphoreType.DMA],
)
def _sc_segsum(src_idx, dst_idx, tab, out,
               sidx2, didx0, didx1, rows0, rows1, zb, ob, acc, sem0, sem1):
    c = lax.axis_index("c")
    s = lax.axis_index("s")
    zero16 = jnp.zeros((16,), _f32)
    for i in range(OCH):
        for k in range(DH // 16):
            zb[i, pl.ds(k * 16, 16)] = zero16

    def zbody(k, carry):
        pltpu.sync_copy(zb, acc.at[pl.ds(s * RPT + k * OCH, OCH)])
        return carry
    lax.fori_loop(0, RPT // OCH, zbody, None)
    plsc.subcore_barrier()

    coff = c * N

    def body(j, carry):
        base = s * PER_TILE + j * (2 * CHUNK)
        pltpu.sync_copy(src_idx.at[pl.ds(base, 2 * CHUNK)], sidx2)
        pltpu.sync_copy(dst_idx.at[pl.ds(base, CHUNK)], didx0)
        pltpu.sync_copy(dst_idx.at[pl.ds(base + CHUNK, CHUNK)], didx1)
        for k in range(2 * CHUNK // 16):
            sidx2[pl.ds(k * 16, 16)] = sidx2[pl.ds(k * 16, 16)] + coff
        g0 = pltpu.async_copy(tab.at[sidx2.at[pl.ds(0, CHUNK)]], rows0, sem0)
        g1 = pltpu.async_copy(tab.at[sidx2.at[pl.ds(CHUNK, CHUNK)]], rows1, sem1)
        g0.wait()
        pltpu.sync_copy(rows0, acc.at[didx0], add=True)
        g1.wait()
        pltpu.sync_copy(rows1, acc.at[didx1], add=True)
        return carry
    lax.fori_loop(0, NCHUNK // 2, body, None)

    tbase = s * PER_TILE + (NCHUNK - 1) * CHUNK
    pltpu.sync_copy(src_idx.at[pl.ds(tbase, CHUNK)], sidx2.at[pl.ds(0, CHUNK)])
    pltpu.sync_copy(dst_idx.at[pl.ds(tbase, CHUNK)], didx0)
    for k in range(CHUNK // 16):
        sidx2[pl.ds(k * 16, 16)] = sidx2[pl.ds(k * 16, 16)] + coff
    pltpu.async_copy(tab.at[sidx2.at[pl.ds(0, CHUNK)]], rows0, sem0).wait()
    pltpu.sync_copy(rows0, acc.at[didx0], add=True)
    plsc.subcore_barrier()

    nch = jnp.minimum(jnp.maximum(N - s * RPT, 0), RPT) // OCH

    def obody(k, carry):
        r = s * RPT + k * OCH
        pltpu.sync_copy(acc.at[pl.ds(r, OCH)], ob)
        pltpu.sync_copy(ob, out.at[pl.ds(c * N + r, OCH)])
        return carry
    lax.fori_loop(0, nch, obody, None)


def _prep_body(dvd, ded, dv, de):
    dv[...] = lax.rsqrt(dvd[...])
    de[...] = 1.0 / ded[...]


def _tc_prep(deg):
    return pl.pallas_call(
        _prep_body,
        grid=(GRID,),
        in_specs=[pl.BlockSpec((BM, DEGW), lambda i: (i, 0)),
                  pl.BlockSpec((BM, DEGW), lambda i: (GRID + i, 0))],
        out_specs=[pl.BlockSpec((BM, DEGW), lambda i: (i, 0)),
                   pl.BlockSpec((BM, DEGW), lambda i: (i, 0))],
        out_shape=[jax.ShapeDtypeStruct((N, DEGW), _f32),
                   jax.ShapeDtypeStruct((N, DEGW), _f32)],
    )(deg, deg)


def _mm1_body(x, w, b, dv, y):
    yy = lax.dot_general(x[...], w[...], (((1,), (1,)), ((), ())),
                         preferred_element_type=_f32)
    y[...] = (yy + b[...]) * dv[...][:, :1]


def _tc_mm1(x, w1, b1r, dv):
    return pl.pallas_call(
        _mm1_body,
        grid=(GRID, 2),
        in_specs=[pl.BlockSpec((BM, D), lambda i, j: (i, 0)),
                  pl.BlockSpec((DH, D), lambda i, j: (j, 0)),
                  pl.BlockSpec((1, DH), lambda i, j: (0, j)),
                  pl.BlockSpec((BM, DEGW), lambda i, j: (i, 0))],
        out_specs=pl.BlockSpec((BM, DH), lambda i, j: (j * GRID + i, 0)),
        out_shape=jax.ShapeDtypeStruct((2 * N, DH), _f32),
    )(x, w1, b1r, dv)


def _scale_body(z, de, o):
    o[...] = z[...] * de[...][:, :1]


def _tc_scale(z, de):
    return pl.pallas_call(
        _scale_body,
        grid=(GRID, 2),
        in_specs=[pl.BlockSpec((BM, DH), lambda i, j: (j * GRID + i, 0)),
                  pl.BlockSpec((BM, DEGW), lambda i, j: (i, 0))],
        out_specs=pl.BlockSpec((BM, DH), lambda i, j: (j * GRID + i, 0)),
        out_shape=jax.ShapeDtypeStruct((2 * N, DH), _f32),
    )(z, de)


def _mid_body(za, zbr, dv, w, b, y):
    d = dv[...][:, :1]
    h = jnp.concatenate([jnp.maximum(za[...] * d, 0.0),
                         jnp.maximum(zbr[...] * d, 0.0)], axis=1)
    yy = lax.dot_general(h, w[...], (((1,), (1,)), ((), ())),
                         preferred_element_type=_f32)
    y[...] = (yy + b[...]) * d


def _tc_mid(zv, dv, w2, b2r):
    return pl.pallas_call(
        _mid_body,
        grid=(GRID, 2),
        in_specs=[pl.BlockSpec((BM, DH), lambda i, j: (i, 0)),
                  pl.BlockSpec((BM, DH), lambda i, j: (GRID + i, 0)),
                  pl.BlockSpec((BM, DEGW), lambda i, j: (i, 0)),
                  pl.BlockSpec((DH, D), lambda i, j: (j, 0)),
                  pl.BlockSpec((1, DH), lambda i, j: (0, j))],
        out_specs=pl.BlockSpec((BM, DH), lambda i, j: (j * GRID + i, 0)),
        out_shape=jax.ShapeDtypeStruct((2 * N, DH), _f32),
    )(zv, zv, dv, w2, b2r)


def _final_body(za, zbr, dv, o):
    d = dv[...][:, :1]
    o[...] = jnp.concatenate([za[...] * d, zbr[...] * d], axis=1)


def _tc_final(zv, dv):
    return pl.pallas_call(
        _final_body,
        grid=(GRID,),
        in_specs=[pl.BlockSpec((BM, DH), lambda i: (i, 0)),
                  pl.BlockSpec((BM, DH), lambda i: (GRID + i, 0)),
                  pl.BlockSpec((BM, DEGW), lambda i: (i, 0))],
        out_specs=pl.BlockSpec((BM, D), lambda i: (i, 0)),
        out_shape=jax.ShapeDtypeStruct((N, D), _f32),
    )(zv, zv, dv)


def kernel(X, W1, b1, W2, b2, node_idx, edge_idx):
    b1r = b1.reshape(1, D)
    b2r = b2.reshape(1, D)
    cidx = jnp.concatenate([node_idx, edge_idx])
    deg = _sc_degrees(cidx)
    dv, de = _tc_prep(deg)
    y1 = _tc_mm1(X, W1, b1r, dv)
    ze = _sc_segsum(node_idx, edge_idx, y1)
    ze = _tc_scale(ze, de)
    zv = _sc_segsum(edge_idx, node_idx, ze)
    y2 = _tc_mid(zv, dv, W2, b2r)
    z2 = _sc_segsum(node_idx, edge_idx, y2)
    z2 = _tc_scale(z2, de)
    z2 = _sc_segsum(edge_idx, node_idx, z2)
    return _tc_final(z2, dv)


# speedup vs baseline: 4.1826x; 1.0932x over previous
import functools

import jax
import jax.numpy as jnp
from jax import lax
from jax.experimental import pallas as pl
from jax.experimental.pallas import tpu as pltpu
from jax.experimental.pallas import tpu_sc as plsc

N = 10000
NNZ = 160000
D = 256
DH = 128
NS = 16
PER_TILE = NNZ // NS
CHUNK = 80
NCHUNK = PER_TILE // CHUNK
NPAD = 10240
RPT = NPAD // NS
OCH = 80
DEGW = 16
BM = 1000
GRID = N // BM

_f32 = jnp.float32
_mesh = plsc.VectorSubcoreMesh(core_axis_name="c", subcore_axis_name="s")


@functools.partial(
    pl.kernel,
    mesh=_mesh,
    out_type=jax.ShapeDtypeStruct((2 * N, DEGW), _f32),
    scratch_types=[pltpu.VMEM((CHUNK,), jnp.int32),
                   pltpu.VMEM((CHUNK,), jnp.int32),
                   pltpu.VMEM((CHUNK, DEGW), _f32),
                   pltpu.VMEM((OCH, DEGW), _f32),
                   pltpu.VMEM((OCH, DEGW), _f32),
                   pltpu.VMEM_SHARED((NPAD, DEGW), _f32),
                   pltpu.SemaphoreType.DMA],
)
def _sc_degrees(cidx, deg_out, idxb0, idxb1, onesb, zb, ob, acc, ssem):
    c = lax.axis_index("c")
    s = lax.axis_index("s")
    ones16 = jnp.ones((16,), _f32)
    zero16 = jnp.zeros((16,), _f32)
    for i in range(CHUNK):
        onesb[i, :] = ones16
    for i in range(OCH):
        zb[i, :] = zero16

    def zbody(k, carry):
        pltpu.sync_copy(zb, acc.at[pl.ds(s * RPT + k * OCH, OCH)])
        return carry
    lax.fori_loop(0, RPT // OCH, zbody, None)
    plsc.subcore_barrier()

    def body(j, carry):
        base = c * NNZ + s * PER_TILE + j * (2 * CHUNK)
        pltpu.sync_copy(cidx.at[pl.ds(base, CHUNK)], idxb0)
        s0 = pltpu.async_copy(onesb, acc.at[idxb0], ssem, add=True)
        pltpu.sync_copy(cidx.at[pl.ds(base + CHUNK, CHUNK)], idxb1)
        s1 = pltpu.async_copy(onesb, acc.at[idxb1], ssem, add=True)
        s0.wait()
        s1.wait()
        return carry
    lax.fori_loop(0, NCHUNK // 2, body, None)

    tbase = c * NNZ + s * PER_TILE + (NCHUNK - 1) * CHUNK
    pltpu.sync_copy(cidx.at[pl.ds(tbase, CHUNK)], idxb0)
    pltpu.sync_copy(onesb, acc.at[idxb0], add=True)
    plsc.subcore_barrier()

    nch = jnp.minimum(jnp.maximum(N - s * RPT, 0), RPT) // OCH

    def obody(k, carry):
        r = s * RPT + k * OCH
        pltpu.sync_copy(acc.at[pl.ds(r, OCH)], ob)
        pltpu.sync_copy(ob, deg_out.at[pl.ds(c * N + r, OCH)])
        return carry
    lax.fori_loop(0, nch, obody, None)


@functools.partial(
    pl.kernel,
    mesh=_mesh,
    out_type=jax.ShapeDtypeStruct((2 * N, DH), _f32),
    scratch_types=[pltpu.VMEM((4 * CHUNK,), jnp.int32),
                   pltpu.VMEM((CHUNK,), jnp.int32),
                   pltpu.VMEM((CHUNK,), jnp.int32),
                   pltpu.VMEM((CHUNK,), jnp.int32),
                   pltpu.VMEM((CHUNK,), jnp.int32),
                   pltpu.VMEM((CHUNK, DH), _f32),
                   pltpu.VMEM((CHUNK, DH), _f32),
                   pltpu.VMEM((CHUNK, DH), _f32),
                   pltpu.VMEM((CHUNK, DH), _f32),
                   pltpu.VMEM_SHARED((NPAD, DH), _f32),
                   pltpu.SemaphoreType.DMA,
                   pltpu.SemaphoreType.DMA,
                   pltpu.SemaphoreType.DMA,
                   pltpu.SemaphoreType.DMA,
                   pltpu.SemaphoreType.DMA],
)
def _sc_segsum(src_idx, dst_idx, tab, out,
               sidx4, didx0, didx1, didx2, didx3,
               rows0, rows1, rows2, rows3, acc,
               gsem0, gsem1, gsem2, gsem3, ssem):
    c = lax.axis_index("c")
    s = lax.axis_index("s")
    zero16 = jnp.zeros((16,), _f32)
    for i in range(OCH):
        for k in range(DH // 16):
            rows0[i, pl.ds(k * 16, 16)] = zero16

    def zbody(k, carry):
        pltpu.sync_copy(rows0, acc.at[pl.ds(s * RPT + k * OCH, OCH)])
        return carry
    lax.fori_loop(0, RPT // OCH, zbody, None)
    plsc.subcore_barrier()

    coff = c * N
    didxs = (didx0, didx1, didx2, didx3)
    rowss = (rows0, rows1, rows2, rows3)
    gsems = (gsem0, gsem1, gsem2, gsem3)

    def body(j, carry):
        base = s * PER_TILE + j * (4 * CHUNK)
        pltpu.sync_copy(src_idx.at[pl.ds(base, 4 * CHUNK)], sidx4)
        for i in range(4):
            pltpu.sync_copy(dst_idx.at[pl.ds(base + i * CHUNK, CHUNK)],
                            didxs[i])
        for k in range(4 * CHUNK // 16):
            sidx4[pl.ds(k * 16, 16)] = sidx4[pl.ds(k * 16, 16)] + coff
        gs = [pltpu.async_copy(tab.at[sidx4.at[pl.ds(i * CHUNK, CHUNK)]],
                               rowss[i], gsems[i]) for i in range(4)]
        scs = []
        for i in range(4):
            gs[i].wait()
            scs.append(pltpu.async_copy(rowss[i], acc.at[didxs[i]], ssem,
                                        add=True))
        for sc in scs:
            sc.wait()
        return carry
    lax.fori_loop(0, NCHUNK // 4, body, None)

    tbase = s * PER_TILE + (NCHUNK - 1) * CHUNK
    pltpu.sync_copy(src_idx.at[pl.ds(tbase, CHUNK)], sidx4.at[pl.ds(0, CHUNK)])
    pltpu.sync_copy(dst_idx.at[pl.ds(tbase, CHUNK)], didx0)
    for k in range(CHUNK // 16):
        sidx4[pl.ds(k * 16, 16)] = sidx4[pl.ds(k * 16, 16)] + coff
    pltpu.async_copy(tab.at[sidx4.at[pl.ds(0, CHUNK)]], rows0, gsem0).wait()
    pltpu.sync_copy(rows0, acc.at[didx0], add=True)
    plsc.subcore_barrier()

    nch = jnp.minimum(jnp.maximum(N - s * RPT, 0), RPT) // OCH

    def obody(k, carry):
        r = s * RPT + k * OCH
        pltpu.sync_copy(acc.at[pl.ds(r, OCH)], rows1)
        pltpu.sync_copy(rows1, out.at[pl.ds(c * N + r, OCH)])
        return carry
    lax.fori_loop(0, nch, obody, None)


def _prep_body(dvd, ded, dv, de):
    dv[...] = lax.rsqrt(dvd[...])
    de[...] = 1.0 / ded[...]


def _tc_prep(deg):
    return pl.pallas_call(
        _prep_body,
        grid=(GRID,),
        in_specs=[pl.BlockSpec((BM, DEGW), lambda i: (i, 0)),
                  pl.BlockSpec((BM, DEGW), lambda i: (GRID + i, 0))],
        out_specs=[pl.BlockSpec((BM, DEGW), lambda i: (i, 0)),
                   pl.BlockSpec((BM, DEGW), lambda i: (i, 0))],
        out_shape=[jax.ShapeDtypeStruct((N, DEGW), _f32),
                   jax.ShapeDtypeStruct((N, DEGW), _f32)],
    )(deg, deg)


def _mm1_body(x, w, b, dv, y):
    yy = lax.dot_general(x[...], w[...], (((1,), (1,)), ((), ())),
                         preferred_element_type=_f32)
    y[...] = (yy + b[...]) * dv[...][:, :1]


def _tc_mm1(x, w1, b1r, dv):
    return pl.pallas_call(
        _mm1_body,
        grid=(GRID, 2),
        in_specs=[pl.BlockSpec((BM, D), lambda i, j: (i, 0)),
                  pl.BlockSpec((DH, D), lambda i, j: (j, 0)),
                  pl.BlockSpec((1, DH), lambda i, j: (0, j)),
                  pl.BlockSpec((BM, DEGW), lambda i, j: (i, 0))],
        out_specs=pl.BlockSpec((BM, DH), lambda i, j: (j * GRID + i, 0)),
        out_shape=jax.ShapeDtypeStruct((2 * N, DH), _f32),
    )(x, w1, b1r, dv)


def _scale_body(z, de, o):
    o[...] = z[...] * de[...][:, :1]


def _tc_scale(z, de):
    return pl.pallas_call(
        _scale_body,
        grid=(GRID, 2),
        in_specs=[pl.BlockSpec((BM, DH), lambda i, j: (j * GRID + i, 0)),
                  pl.BlockSpec((BM, DEGW), lambda i, j: (i, 0))],
        out_specs=pl.BlockSpec((BM, DH), lambda i, j: (j * GRID + i, 0)),
        out_shape=jax.ShapeDtypeStruct((2 * N, DH), _f32),
    )(z, de)


def _mid_body(za, zbr, dv, w, b, y):
    d = dv[...][:, :1]
    h = jnp.concatenate([jnp.maximum(za[...] * d, 0.0),
                         jnp.maximum(zbr[...] * d, 0.0)], axis=1)
    yy = lax.dot_general(h, w[...], (((1,), (1,)), ((), ())),
                         preferred_element_type=_f32)
    y[...] = (yy + b[...]) * d


def _tc_mid(zv, dv, w2, b2r):
    return pl.pallas_call(
        _mid_body,
        grid=(GRID, 2),
        in_specs=[pl.BlockSpec((BM, DH), lambda i, j: (i, 0)),
                  pl.BlockSpec((BM, DH), lambda i, j: (GRID + i, 0)),
                  pl.BlockSpec((BM, DEGW), lambda i, j: (i, 0)),
                  pl.BlockSpec((DH, D), lambda i, j: (j, 0)),
                  pl.BlockSpec((1, DH), lambda i, j: (0, j))],
        out_specs=pl.BlockSpec((BM, DH), lambda i, j: (j * GRID + i, 0)),
        out_shape=jax.ShapeDtypeStruct((2 * N, DH), _f32),
    )(zv, zv, dv, w2, b2r)


def _final_body(za, zbr, dv, o):
    d = dv[...][:, :1]
    o[...] = jnp.concatenate([za[...] * d, zbr[...] * d], axis=1)


def _tc_final(zv, dv):
    return pl.pallas_call(
        _final_body,
        grid=(GRID,),
        in_specs=[pl.BlockSpec((BM, DH), lambda i: (i, 0)),
                  pl.BlockSpec((BM, DH), lambda i: (GRID + i, 0)),
                  pl.BlockSpec((BM, DEGW), lambda i: (i, 0))],
        out_specs=pl.BlockSpec((BM, D), lambda i: (i, 0)),
        out_shape=jax.ShapeDtypeStruct((N, D), _f32),
    )(zv, zv, dv)


def kernel(X, W1, b1, W2, b2, node_idx, edge_idx):
    b1r = b1.reshape(1, D)
    b2r = b2.reshape(1, D)
    cidx = jnp.concatenate([node_idx, edge_idx])
    deg = _sc_degrees(cidx)
    dv, de = _tc_prep(deg)
    y1 = _tc_mm1(X, W1, b1r, dv)
    ze = _sc_segsum(node_idx, edge_idx, y1)
    ze = _tc_scale(ze, de)
    zv = _sc_segsum(edge_idx, node_idx, ze)
    y2 = _tc_mid(zv, dv, W2, b2r)
    z2 = _sc_segsum(node_idx, edge_idx, y2)
    z2 = _tc_scale(z2, de)
    z2 = _sc_segsum(edge_idx, node_idx, z2)
    return _tc_final(z2, dv)


# speedup vs baseline: 6.5438x; 1.5645x over previous
import functools

import jax
import jax.numpy as jnp
from jax import lax
from jax.experimental import pallas as pl
from jax.experimental.pallas import tpu as pltpu
from jax.experimental.pallas import tpu_sc as plsc

N = 10000
NNZ = 160000
D = 256
DH = 128
NS = 16
PER_TILE = NNZ // NS
CHUNK = 40
K = 5
BODY = K * CHUNK
NBODY = PER_TILE // BODY
NROW3 = NNZ // BODY
NPAD = 10240
RPT = NPAD // NS
OCH = 80
DEGW = 16
DCH = 80
DNCH = PER_TILE // DCH
BM = 1000
GRID = N // BM

_f32 = jnp.float32
_mesh = plsc.VectorSubcoreMesh(core_axis_name="c", subcore_axis_name="s")


@functools.partial(
    pl.kernel,
    mesh=_mesh,
    out_type=jax.ShapeDtypeStruct((2 * N, DEGW), _f32),
    scratch_types=[pltpu.VMEM((DCH,), jnp.int32),
                   pltpu.VMEM((DCH,), jnp.int32),
                   pltpu.VMEM((DCH, DEGW), _f32),
                   pltpu.VMEM((OCH, DEGW), _f32),
                   pltpu.VMEM((OCH, DEGW), _f32),
                   pltpu.VMEM_SHARED((NPAD, DEGW), _f32),
                   pltpu.SemaphoreType.DMA],
)
def _sc_degrees(cidx, deg_out, idxb0, idxb1, onesb, zb, ob, acc, ssem):
    c = lax.axis_index("c")
    s = lax.axis_index("s")
    ones16 = jnp.ones((16,), _f32)
    zero16 = jnp.zeros((16,), _f32)
    for i in range(DCH):
        onesb[i, :] = ones16
    for i in range(OCH):
        zb[i, :] = zero16

    def zbody(k, carry):
        pltpu.sync_copy(zb, acc.at[pl.ds(s * RPT + k * OCH, OCH)])
        return carry
    lax.fori_loop(0, RPT // OCH, zbody, None)
    plsc.subcore_barrier()

    def body(j, carry):
        base = c * NNZ + s * PER_TILE + j * (2 * DCH)
        pltpu.sync_copy(cidx.at[pl.ds(base, DCH)], idxb0)
        s0 = pltpu.async_copy(onesb, acc.at[idxb0], ssem, add=True)
        pltpu.sync_copy(cidx.at[pl.ds(base + DCH, DCH)], idxb1)
        s1 = pltpu.async_copy(onesb, acc.at[idxb1], ssem, add=True)
        s0.wait()
        s1.wait()
        return carry
    lax.fori_loop(0, DNCH // 2, body, None)

    tbase = c * NNZ + s * PER_TILE + (DNCH - 1) * DCH
    pltpu.sync_copy(cidx.at[pl.ds(tbase, DCH)], idxb0)
    pltpu.sync_copy(onesb, acc.at[idxb0], add=True)
    plsc.subcore_barrier()

    nch = jnp.minimum(jnp.maximum(N - s * RPT, 0), RPT) // OCH

    def obody(k, carry):
        r = s * RPT + k * OCH
        pltpu.sync_copy(acc.at[pl.ds(r, OCH)], ob)
        pltpu.sync_copy(ob, deg_out.at[pl.ds(c * N + r, OCH)])
        return carry
    lax.fori_loop(0, nch, obody, None)


@functools.partial(
    pl.kernel,
    mesh=_mesh,
    out_type=jax.ShapeDtypeStruct((2 * N, DH), _f32),
    scratch_types=[pltpu.VMEM((BODY,), jnp.int32),
                   pltpu.VMEM((BODY,), jnp.int32),
                   pltpu.VMEM((K, CHUNK), jnp.int32),
                   pltpu.VMEM((K, CHUNK), jnp.int32),
                   pltpu.VMEM((CHUNK, DH), _f32),
                   pltpu.VMEM((CHUNK, DH), _f32),
                   pltpu.VMEM((CHUNK, DH), _f32),
                   pltpu.VMEM((CHUNK, DH), _f32),
                   pltpu.VMEM((CHUNK, DH), _f32),
                   pltpu.VMEM((OCH, DH), _f32),
                   pltpu.VMEM_SHARED((NPAD, DH), _f32),
                   pltpu.SemaphoreType.DMA,
                   pltpu.SemaphoreType.DMA,
                   pltpu.SemaphoreType.DMA,
                   pltpu.SemaphoreType.DMA,
                   pltpu.SemaphoreType.DMA,
                   pltpu.SemaphoreType.DMA,
                   pltpu.SemaphoreType.DMA,
                   pltpu.SemaphoreType.DMA],
)
def _sc_segsum(src2, dst3, tab, out,
               sidxA, sidxB, didxA, didxB,
               rows0, rows1, rows2, rows3, rows4, stg, acc,
               g0, g1, g2, g3, g4, ssem, isemA, isemB):
    c = lax.axis_index("c")
    s = lax.axis_index("s")
    rows = (rows0, rows1, rows2, rows3, rows4)
    gsems = (g0, g1, g2, g3, g4)
    zero16 = jnp.zeros((16,), _f32)
    for i in range(OCH):
        for k in range(DH // 16):
            stg[i, pl.ds(k * 16, 16)] = zero16

    def zbody(k, carry):
        pltpu.sync_copy(stg, acc.at[pl.ds(s * RPT + k * OCH, OCH)])
        return carry
    lax.fori_loop(0, RPT // OCH, zbody, None)
    plsc.subcore_barrier()

    sbase = c * NNZ + s * PER_TILE
    rbase = s * NBODY

    pltpu.sync_copy(src2.at[pl.ds(sbase, BODY)], sidxA)
    pltpu.sync_copy(dst3.at[rbase], didxA)

    def body(j, carry):
        jA, jB, jA2 = 2 * j, 2 * j + 1, 2 * j + 2
        jA2c = jnp.minimum(jA2, NBODY - 1)

        gA = [pltpu.async_copy(tab.at[sidxA.at[pl.ds(i * CHUNK, CHUNK)]],
                               rows[i], gsems[i]) for i in range(K)]
        iB0 = pltpu.async_copy(src2.at[pl.ds(sbase + jB * BODY, BODY)],
                               sidxB, isemB)
        iB1 = pltpu.async_copy(dst3.at[rbase + jB], didxB, isemB)
        sA = []
        for i in range(K):
            gA[i].wait()
            sA.append(pltpu.async_copy(rows[i], acc.at[didxA.at[i]], ssem,
                                       add=True))
        iA0 = pltpu.async_copy(src2.at[pl.ds(sbase + jA2c * BODY, BODY)],
                               sidxA, isemA)
        iB0.wait()
        iB1.wait()
        gB = []
        for i in range(K):
            sA[i].wait()
            gB.append(pltpu.async_copy(tab.at[sidxB.at[pl.ds(i * CHUNK,
                                                             CHUNK)]],
                                       rows[i], gsems[i]))
        iA1 = pltpu.async_copy(dst3.at[rbase + jA2c], didxA, isemA)
        sB = []
        for i in range(K):
            gB[i].wait()
            sB.append(pltpu.async_copy(rows[i], acc.at[didxB.at[i]], ssem,
                                       add=True))
        for i in range(K):
            sB[i].wait()
        iA0.wait()
        iA1.wait()
        return carry
    lax.fori_loop(0, NBODY // 2, body, None)
    plsc.subcore_barrier()

    nch = jnp.minimum(jnp.maximum(N - s * RPT, 0), RPT) // OCH

    def obody(k, carry):
        r = s * RPT + k * OCH
        pltpu.sync_copy(acc.at[pl.ds(r, OCH)], stg)
        pltpu.sync_copy(stg, out.at[pl.ds(c * N + r, OCH)])
        return carry
    lax.fori_loop(0, nch, obody, None)


def _prep_body(dvd, ded, dv, de):
    dv[...] = lax.rsqrt(dvd[...])
    de[...] = 1.0 / ded[...]


def _tc_prep(deg):
    return pl.pallas_call(
        _prep_body,
        grid=(GRID,),
        in_specs=[pl.BlockSpec((BM, DEGW), lambda i: (i, 0)),
                  pl.BlockSpec((BM, DEGW), lambda i: (GRID + i, 0))],
        out_specs=[pl.BlockSpec((BM, DEGW), lambda i: (i, 0)),
                   pl.BlockSpec((BM, DEGW), lambda i: (i, 0))],
        out_shape=[jax.ShapeDtypeStruct((N, DEGW), _f32),
                   jax.ShapeDtypeStruct((N, DEGW), _f32)],
    )(deg, deg)


def _mm1_body(x, w, b, dv, y):
    yy = lax.dot_general(x[...], w[...], (((1,), (1,)), ((), ())),
                         preferred_element_type=_f32)
    y[...] = (yy + b[...]) * dv[...][:, :1]


def _tc_mm1(x, w1, b1r, dv):
    return pl.pallas_call(
        _mm1_body,
        grid=(GRID, 2),
        in_specs=[pl.BlockSpec((BM, D), lambda i, j: (i, 0)),
                  pl.BlockSpec((DH, D), lambda i, j: (j, 0)),
                  pl.BlockSpec((1, DH), lambda i, j: (0, j)),
                  pl.BlockSpec((BM, DEGW), lambda i, j: (i, 0))],
        out_specs=pl.BlockSpec((BM, DH), lambda i, j: (j * GRID + i, 0)),
        out_shape=jax.ShapeDtypeStruct((2 * N, DH), _f32),
    )(x, w1, b1r, dv)


def _scale_body(z, de, o):
    o[...] = z[...] * de[...][:, :1]


def _tc_scale(z, de):
    return pl.pallas_call(
        _scale_body,
        grid=(GRID, 2),
        in_specs=[pl.BlockSpec((BM, DH), lambda i, j: (j * GRID + i, 0)),
                  pl.BlockSpec((BM, DEGW), lambda i, j: (i, 0))],
        out_specs=pl.BlockSpec((BM, DH), lambda i, j: (j * GRID + i, 0)),
        out_shape=jax.ShapeDtypeStruct((2 * N, DH), _f32),
    )(z, de)


def _mid_body(za, zbr, dv, w, b, y):
    d = dv[...][:, :1]
    h = jnp.concatenate([jnp.maximum(za[...] * d, 0.0),
                         jnp.maximum(zbr[...] * d, 0.0)], axis=1)
    yy = lax.dot_general(h, w[...], (((1,), (1,)), ((), ())),
                         preferred_element_type=_f32)
    y[...] = (yy + b[...]) * d


def _tc_mid(zv, dv, w2, b2r):
    return pl.pallas_call(
        _mid_body,
        grid=(GRID, 2),
        in_specs=[pl.BlockSpec((BM, DH), lambda i, j: (i, 0)),
                  pl.BlockSpec((BM, DH), lambda i, j: (GRID + i, 0)),
                  pl.BlockSpec((BM, DEGW), lambda i, j: (i, 0)),
                  pl.BlockSpec((DH, D), lambda i, j: (j, 0)),
                  pl.BlockSpec((1, DH), lambda i, j: (0, j))],
        out_specs=pl.BlockSpec((BM, DH), lambda i, j: (j * GRID + i, 0)),
        out_shape=jax.ShapeDtypeStruct((2 * N, DH), _f32),
    )(zv, zv, dv, w2, b2r)


def _final_body(za, zbr, dv, o):
    d = dv[...][:, :1]
    o[...] = jnp.concatenate([za[...] * d, zbr[...] * d], axis=1)


def _tc_final(zv, dv):
    return pl.pallas_call(
        _final_body,
        grid=(GRID,),
        in_specs=[pl.BlockSpec((BM, DH), lambda i: (i, 0)),
                  pl.BlockSpec((BM, DH), lambda i: (GRID + i, 0)),
                  pl.BlockSpec((BM, DEGW), lambda i: (i, 0))],
        out_specs=pl.BlockSpec((BM, D), lambda i: (i, 0)),
        out_shape=jax.ShapeDtypeStruct((N, D), _f32),
    )(zv, zv, dv)


def kernel(X, W1, b1, W2, b2, node_idx, edge_idx):
    b1r = b1.reshape(1, D)
    b2r = b2.reshape(1, D)
    cidx = jnp.concatenate([node_idx, edge_idx])
    s_node = jnp.concatenate([node_idx, node_idx + N])
    s_edge = jnp.concatenate([edge_idx, edge_idx + N])
    d_node = node_idx.reshape(NROW3, K, CHUNK)
    d_edge = edge_idx.reshape(NROW3, K, CHUNK)
    deg = _sc_degrees(cidx)
    dv, de = _tc_prep(deg)
    y1 = _tc_mm1(X, W1, b1r, dv)
    ze = _sc_segsum(s_node, d_edge, y1)
    ze = _tc_scale(ze, de)
    zv = _sc_segsum(s_edge, d_node, ze)
    y2 = _tc_mid(zv, dv, W2, b2r)
    z2 = _sc_segsum(s_node, d_edge, y2)
    z2 = _tc_scale(z2, de)
    z2 = _sc_segsum(s_edge, d_node, z2)
    return _tc_final(z2, dv)


# speedup vs baseline: 7.1685x; 1.0955x over previous
import functools

import jax
import jax.numpy as jnp
from jax import lax
from jax.experimental import pallas as pl
from jax.experimental.pallas import tpu as pltpu
from jax.experimental.pallas import tpu_sc as plsc

N = 10000
NNZ = 160000
D = 256
DH = 128
NS = 16
PER_TILE = NNZ // NS
CHUNK = 40
K = 5
BODY = K * CHUNK
NBODY = PER_TILE // BODY
NROW3 = NNZ // BODY
NPAD = 10240
RPT = NPAD // NS
OCH = 80
DEGW = 16
DCH = 125
DK = 4
DNB = PER_TILE // (DK * DCH)
DROW3 = 2 * NNZ // (DK * DCH)
BM = 1000
GRID = N // BM

_f32 = jnp.float32
_mesh = plsc.VectorSubcoreMesh(core_axis_name="c", subcore_axis_name="s")


@functools.partial(
    pl.kernel,
    mesh=_mesh,
    out_type=jax.ShapeDtypeStruct((2 * N, DEGW), _f32),
    scratch_types=[pltpu.VMEM((DK, DCH), jnp.int32),
                   pltpu.VMEM((DK, DCH), jnp.int32),
                   pltpu.VMEM((DCH, DEGW), _f32),
                   pltpu.VMEM((OCH, DEGW), _f32),
                   pltpu.VMEM((OCH, DEGW), _f32),
                   pltpu.VMEM_SHARED((NPAD, DEGW), _f32),
                   pltpu.SemaphoreType.DMA,
                   pltpu.SemaphoreType.DMA,
                   pltpu.SemaphoreType.DMA],
)
def _sc_degrees(cidx3, deg_out, didxA, didxB, onesb, zb, ob, acc,
                ssem, isemA, isemB):
    c = lax.axis_index("c")
    s = lax.axis_index("s")
    ones16 = jnp.ones((16,), _f32)
    zero16 = jnp.zeros((16,), _f32)
    for i in range(DCH):
        onesb[i, :] = ones16
    for i in range(OCH):
        zb[i, :] = zero16

    def zbody(k, carry):
        pltpu.sync_copy(zb, acc.at[pl.ds(s * RPT + k * OCH, OCH)])
        return carry
    lax.fori_loop(0, RPT // OCH, zbody, None)
    plsc.subcore_barrier()

    rb = c * (DROW3 // 2) + s * DNB
    pltpu.sync_copy(cidx3.at[rb], didxA)

    def body(j, carry):
        rA2 = rb + jnp.minimum(2 * j + 2, DNB - 1)
        sA = [pltpu.async_copy(onesb, acc.at[didxA.at[i]], ssem, add=True)
              for i in range(DK)]
        iB = pltpu.async_copy(cidx3.at[rb + 2 * j + 1], didxB, isemB)
        for i in range(DK):
            sA[i].wait()
        iA = pltpu.async_copy(cidx3.at[rA2], didxA, isemA)
        iB.wait()
        sB = [pltpu.async_copy(onesb, acc.at[didxB.at[i]], ssem, add=True)
              for i in range(DK)]
        for i in range(DK):
            sB[i].wait()
        iA.wait()
        return carry
    lax.fori_loop(0, DNB // 2, body, None)
    plsc.subcore_barrier()

    nch = jnp.minimum(jnp.maximum(N - s * RPT, 0), RPT) // OCH

    def obody(k, carry):
        r = s * RPT + k * OCH
        pltpu.sync_copy(acc.at[pl.ds(r, OCH)], ob)
        pltpu.sync_copy(ob, deg_out.at[pl.ds(c * N + r, OCH)])
        return carry
    lax.fori_loop(0, nch, obody, None)


def _seg_impl(src2, dst3, tab, sc16, out,
              sidxA, sidxB, didxA, didxB,
              rows0, rows1, rows2, rows3, rows4, stg, scb, acc,
              g0, g1, g2, g3, g4, ssem, isemA, isemB, scaled):
    c = lax.axis_index("c")
    s = lax.axis_index("s")
    rows = (rows0, rows1, rows2, rows3, rows4)
    gsems = (g0, g1, g2, g3, g4)
    zero16 = jnp.zeros((16,), _f32)
    for i in range(OCH):
        for k in range(DH // 16):
            stg[i, pl.ds(k * 16, 16)] = zero16

    def zbody(k, carry):
        pltpu.sync_copy(stg, acc.at[pl.ds(s * RPT + k * OCH, OCH)])
        return carry
    lax.fori_loop(0, RPT // OCH, zbody, None)
    plsc.subcore_barrier()

    sbase = c * NNZ + s * PER_TILE
    rbase = s * NBODY

    pltpu.sync_copy(src2.at[pl.ds(sbase, BODY)], sidxA)
    pltpu.sync_copy(dst3.at[rbase], didxA)

    def body(j, carry):
        jA, jB, jA2 = 2 * j, 2 * j + 1, 2 * j + 2
        jA2c = jnp.minimum(jA2, NBODY - 1)

        gA = [pltpu.async_copy(tab.at[sidxA.at[pl.ds(i * CHUNK, CHUNK)]],
                               rows[i], gsems[i]) for i in range(K)]
        iB0 = pltpu.async_copy(src2.at[pl.ds(sbase + jB * BODY, BODY)],
                               sidxB, isemB)
        iB1 = pltpu.async_copy(dst3.at[rbase + jB], didxB, isemB)
        sA = []
        for i in range(K):
            gA[i].wait()
            sA.append(pltpu.async_copy(rows[i], acc.at[didxA.at[i]], ssem,
                                       add=True))
        iA0 = pltpu.async_copy(src2.at[pl.ds(sbase + jA2c * BODY, BODY)],
                               sidxA, isemA)
        iB0.wait()
        iB1.wait()
        gB = []
        for i in range(K):
            sA[i].wait()
            gB.append(pltpu.async_copy(tab.at[sidxB.at[pl.ds(i * CHUNK,
                                                             CHUNK)]],
                                       rows[i], gsems[i]))
        iA1 = pltpu.async_copy(dst3.at[rbase + jA2c], didxA, isemA)
        sB = []
        for i in range(K):
            gB[i].wait()
            sB.append(pltpu.async_copy(rows[i], acc.at[didxB.at[i]], ssem,
                                       add=True))
        for i in range(K):
            sB[i].wait()
        iA0.wait()
        iA1.wait()
        return carry
    lax.fori_loop(0, NBODY // 2, body, None)
    plsc.subcore_barrier()

    nch = jnp.minimum(jnp.maximum(N - s * RPT, 0), RPT) // OCH

    def obody(k, carry):
        r = s * RPT + k * OCH
        pltpu.sync_copy(acc.at[pl.ds(r, OCH)], stg)
        if scaled:
            pltpu.sync_copy(sc16.at[pl.ds(r, OCH)], scb)
            for i in range(OCH):
                v = scb[i, :]
                for k2 in range(DH // 16):
                    stg[i, pl.ds(k2 * 16, 16)] = (
                        stg[i, pl.ds(k2 * 16, 16)] * v)
        pltpu.sync_copy(stg, out.at[pl.ds(c * N + r, OCH)])
        return carry
    lax.fori_loop(0, nch, obody, None)


_SEG_SCRATCH = ([pltpu.VMEM((BODY,), jnp.int32),
                 pltpu.VMEM((BODY,), jnp.int32),
                 pltpu.VMEM((K, CHUNK), jnp.int32),
                 pltpu.VMEM((K, CHUNK), jnp.int32)]
                + [pltpu.VMEM((CHUNK, DH), _f32)] * 5
                + [pltpu.VMEM((OCH, DH), _f32)])
_SEG_SEMS = [pltpu.SemaphoreType.DMA] * 8


@functools.partial(
    pl.kernel,
    mesh=_mesh,
    out_type=jax.ShapeDtypeStruct((2 * N, DH), _f32),
    scratch_types=(_SEG_SCRATCH
                   + [pltpu.VMEM_SHARED((NPAD, DH), _f32)]
                   + _SEG_SEMS),
)
def _sc_segsum(src2, dst3, tab, out,
               sidxA, sidxB, didxA, didxB,
               rows0, rows1, rows2, rows3, rows4, stg, acc,
               g0, g1, g2, g3, g4, ssem, isemA, isemB):
    _seg_impl(src2, dst3, tab, None, out,
              sidxA, sidxB, didxA, didxB,
              rows0, rows1, rows2, rows3, rows4, stg, None, acc,
              g0, g1, g2, g3, g4, ssem, isemA, isemB, scaled=False)


@functools.partial(
    pl.kernel,
    mesh=_mesh,
    out_type=jax.ShapeDtypeStruct((2 * N, DH), _f32),
    scratch_types=(_SEG_SCRATCH
                   + [pltpu.VMEM((OCH, DEGW), _f32)]
                   + [pltpu.VMEM_SHARED((NPAD, DH), _f32)]
                   + _SEG_SEMS),
)
def _sc_segsum_scaled(src2, dst3, tab, sc16, out,
                      sidxA, sidxB, didxA, didxB,
                      rows0, rows1, rows2, rows3, rows4, stg, scb, acc,
                      g0, g1, g2, g3, g4, ssem, isemA, isemB):
    _seg_impl(src2, dst3, tab, sc16, out,
              sidxA, sidxB, didxA, didxB,
              rows0, rows1, rows2, rows3, rows4, stg, scb, acc,
              g0, g1, g2, g3, g4, ssem, isemA, isemB, scaled=True)


def _mm1_body(x, w, b, y):
    yy = lax.dot_general(x[...], w[...], (((1,), (1,)), ((), ())),
                         preferred_element_type=_f32)
    y[...] = yy + b[...]


def _tc_mm1(x, w1, b1r):
    return pl.pallas_call(
        _mm1_body,
        grid=(GRID, 2),
        in_specs=[pl.BlockSpec((BM, D), lambda i, j: (i, 0)),
                  pl.BlockSpec((DH, D), lambda i, j: (j, 0)),
                  pl.BlockSpec((1, DH), lambda i, j: (0, j))],
        out_specs=pl.BlockSpec((BM, DH), lambda i, j: (j * GRID + i, 0)),
        out_shape=jax.ShapeDtypeStruct((2 * N, DH), _f32),
    )(x, w1, b1r)


def _prep_body(dvd, ded, y1p, dv, de, y1):
    d = lax.rsqrt(dvd[...])
    dv[...] = d
    de[...] = 1.0 / ded[...]
    y1[...] = y1p[...] * d[:, :1]


def _tc_prep(deg, y1p):
    return pl.pallas_call(
        _prep_body,
        grid=(GRID, 2),
        in_specs=[pl.BlockSpec((BM, DEGW), lambda i, j: (i, 0)),
                  pl.BlockSpec((BM, DEGW), lambda i, j: (GRID + i, 0)),
                  pl.BlockSpec((BM, DH), lambda i, j: (j * GRID + i, 0))],
        out_specs=[pl.BlockSpec((BM, DEGW), lambda i, j: (i, 0)),
                   pl.BlockSpec((BM, DEGW), lambda i, j: (i, 0)),
                   pl.BlockSpec((BM, DH), lambda i, j: (j * GRID + i, 0))],
        out_shape=[jax.ShapeDtypeStruct((N, DEGW), _f32),
                   jax.ShapeDtypeStruct((N, DEGW), _f32),
                   jax.ShapeDtypeStruct((2 * N, DH), _f32)],
    )(deg, deg, y1p)


def _mid_body(za, zbr, dv, w, b, y):
    d = dv[...][:, :1]
    h = jnp.concatenate([jnp.maximum(za[...] * d, 0.0),
                         jnp.maximum(zbr[...] * d, 0.0)], axis=1)
    yy = lax.dot_general(h, w[...], (((1,), (1,)), ((), ())),
                         preferred_element_type=_f32)
    y[...] = (yy + b[...]) * d


def _tc_mid(zv, dv, w2, b2r):
    return pl.pallas_call(
        _mid_body,
        grid=(GRID, 2),
        in_specs=[pl.BlockSpec((BM, DH), lambda i, j: (i, 0)),
                  pl.BlockSpec((BM, DH), lambda i, j: (GRID + i, 0)),
                  pl.BlockSpec((BM, DEGW), lambda i, j: (i, 0)),
                  pl.BlockSpec((DH, D), lambda i, j: (j, 0)),
                  pl.BlockSpec((1, DH), lambda i, j: (0, j))],
        out_specs=pl.BlockSpec((BM, DH), lambda i, j: (j * GRID + i, 0)),
        out_shape=jax.ShapeDtypeStruct((2 * N, DH), _f32),
    )(zv, zv, dv, w2, b2r)


def _final_body(za, zbr, dv, o):
    d = dv[...][:, :1]
    o[...] = jnp.concatenate([za[...] * d, zbr[...] * d], axis=1)


def _tc_final(zv, dv):
    return pl.pallas_call(
        _final_body,
        grid=(GRID,),
        in_specs=[pl.BlockSpec((BM, DH), lambda i: (i, 0)),
                  pl.BlockSpec((BM, DH), lambda i: (GRID + i, 0)),
                  pl.BlockSpec((BM, DEGW), lambda i: (i, 0))],
        out_specs=pl.BlockSpec((BM, D), lambda i: (i, 0)),
        out_shape=jax.ShapeDtypeStruct((N, D), _f32),
    )(zv, zv, dv)


def kernel(X, W1, b1, W2, b2, node_idx, edge_idx):
    b1r = b1.reshape(1, D)
    b2r = b2.reshape(1, D)
    cidx3 = jnp.concatenate([node_idx, edge_idx]).reshape(DROW3, DK, DCH)
    s_node = jnp.concatenate([node_idx, node_idx + N])
    s_edge = jnp.concatenate([edge_idx, edge_idx + N])
    d_node = node_idx.reshape(NROW3, K, CHUNK)
    d_edge = edge_idx.reshape(NROW3, K, CHUNK)
    y1p = _tc_mm1(X, W1, b1r)
    deg = _sc_degrees(cidx3)
    dv, de, y1 = _tc_prep(deg, y1p)
    ze = _sc_segsum_scaled(s_node, d_edge, y1, de)
    zv = _sc_segsum(s_edge, d_node, ze)
    y2 = _tc_mid(zv, dv, W2, b2r)
    z2 = _sc_segsum_scaled(s_node, d_edge, y2, de)
    z2 = _sc_segsum(s_edge, d_node, z2)
    return _tc_final(z2, dv)


# speedup vs baseline: 7.2596x; 1.0127x over previous
import functools

import jax
import jax.numpy as jnp
from jax import lax
from jax.experimental import pallas as pl
from jax.experimental.pallas import tpu as pltpu
from jax.experimental.pallas import tpu_sc as plsc

N = 10000
NNZ = 160000
D = 256
DH = 128
NS = 16
PER_TILE = NNZ // NS
CHUNK = 40
K = 5
BODY = K * CHUNK
NBODY = PER_TILE // BODY
NROW3 = NNZ // BODY
NPAD = 10240
RPT = NPAD // NS
OCH = 80
DEGW = 16
DCH = 125
DK = 4
DNB = PER_TILE // (DK * DCH)
DROW3 = 2 * NNZ // (DK * DCH)
BM = 1000
GRID = N // BM

_f32 = jnp.float32
_mesh = plsc.VectorSubcoreMesh(core_axis_name="c", subcore_axis_name="s")


@functools.partial(
    pl.kernel,
    mesh=_mesh,
    out_type=jax.ShapeDtypeStruct((2 * N, DEGW), _f32),
    scratch_types=[pltpu.VMEM((DK, DCH), jnp.int32),
                   pltpu.VMEM((DK, DCH), jnp.int32),
                   pltpu.VMEM((DCH, DEGW), _f32),
                   pltpu.VMEM((OCH, DEGW), _f32),
                   pltpu.VMEM((OCH, DEGW), _f32),
                   pltpu.VMEM_SHARED((NPAD, DEGW), _f32),
                   pltpu.SemaphoreType.DMA,
                   pltpu.SemaphoreType.DMA,
                   pltpu.SemaphoreType.DMA],
)
def _sc_degrees(cidx3, deg_out, didxA, didxB, onesb, zb, ob, acc,
                ssem, isemA, isemB):
    c = lax.axis_index("c")
    s = lax.axis_index("s")
    ones16 = jnp.ones((16,), _f32)
    zero16 = jnp.zeros((16,), _f32)
    for i in range(DCH):
        onesb[i, :] = ones16
    for i in range(OCH):
        zb[i, :] = zero16

    zcs = [pltpu.async_copy(zb, acc.at[pl.ds(s * RPT + k * OCH, OCH)],
                            isemA) for k in range(RPT // OCH)]
    for zc in zcs:
        zc.wait()
    plsc.subcore_barrier()

    rb = c * (DROW3 // 2) + s * DNB
    pltpu.sync_copy(cidx3.at[rb], didxA)

    def body(j, carry):
        rA2 = rb + jnp.minimum(2 * j + 2, DNB - 1)
        sA = [pltpu.async_copy(onesb, acc.at[didxA.at[i]], ssem, add=True)
              for i in range(DK)]
        iB = pltpu.async_copy(cidx3.at[rb + 2 * j + 1], didxB, isemB)
        for i in range(DK):
            sA[i].wait()
        iA = pltpu.async_copy(cidx3.at[rA2], didxA, isemA)
        iB.wait()
        sB = [pltpu.async_copy(onesb, acc.at[didxB.at[i]], ssem, add=True)
              for i in range(DK)]
        for i in range(DK):
            sB[i].wait()
        iA.wait()
        return carry
    lax.fori_loop(0, DNB // 2, body, None)
    plsc.subcore_barrier()

    nch = jnp.minimum(jnp.maximum(N - s * RPT, 0), RPT) // OCH

    def obody(k, carry):
        r = s * RPT + k * OCH
        pltpu.sync_copy(acc.at[pl.ds(r, OCH)], ob)
        pltpu.sync_copy(ob, deg_out.at[pl.ds(c * N + r, OCH)])
        return carry
    lax.fori_loop(0, nch, obody, None)


def _seg_impl(src2, dst3, tab, sc16, out,
              sidxA, sidxB, didxA, didxB,
              rows0, rows1, rows2, rows3, rows4, stg, scb, acc,
              g0, g1, g2, g3, g4, ssem, isemA, isemB, scaled,
              concat_out=False):
    c = lax.axis_index("c")
    s = lax.axis_index("s")
    rows = (rows0, rows1, rows2, rows3, rows4)
    gsems = (g0, g1, g2, g3, g4)
    zero16 = jnp.zeros((16,), _f32)
    for i in range(OCH):
        for k in range(DH // 16):
            stg[i, pl.ds(k * 16, 16)] = zero16

    zcs = [pltpu.async_copy(stg, acc.at[pl.ds(s * RPT + k * OCH, OCH)],
                            isemA) for k in range(RPT // OCH)]
    for zc in zcs:
        zc.wait()
    plsc.subcore_barrier()

    sbase = c * NNZ + s * PER_TILE
    rbase = s * NBODY

    pltpu.sync_copy(src2.at[pl.ds(sbase, BODY)], sidxA)
    pltpu.sync_copy(dst3.at[rbase], didxA)

    def body(j, carry):
        jA, jB, jA2 = 2 * j, 2 * j + 1, 2 * j + 2
        jA2c = jnp.minimum(jA2, NBODY - 1)

        gA = [pltpu.async_copy(tab.at[sidxA.at[pl.ds(i * CHUNK, CHUNK)]],
                               rows[i], gsems[i]) for i in range(K)]
        iB0 = pltpu.async_copy(src2.at[pl.ds(sbase + jB * BODY, BODY)],
                               sidxB, isemB)
        iB1 = pltpu.async_copy(dst3.at[rbase + jB], didxB, isemB)
        sA = []
        for i in range(K):
            gA[i].wait()
            sA.append(pltpu.async_copy(rows[i], acc.at[didxA.at[i]], ssem,
                                       add=True))
        iA0 = pltpu.async_copy(src2.at[pl.ds(sbase + jA2c * BODY, BODY)],
                               sidxA, isemA)
        iB0.wait()
        iB1.wait()
        gB = []
        for i in range(K):
            sA[i].wait()
            gB.append(pltpu.async_copy(tab.at[sidxB.at[pl.ds(i * CHUNK,
                                                             CHUNK)]],
                                       rows[i], gsems[i]))
        iA1 = pltpu.async_copy(dst3.at[rbase + jA2c], didxA, isemA)
        sB = []
        for i in range(K):
            gB[i].wait()
            sB.append(pltpu.async_copy(rows[i], acc.at[didxB.at[i]], ssem,
                                       add=True))
        for i in range(K):
            sB[i].wait()
        iA0.wait()
        iA1.wait()
        return carry
    lax.fori_loop(0, NBODY // 2, body, None)
    plsc.subcore_barrier()

    nch = jnp.minimum(jnp.maximum(N - s * RPT, 0), RPT) // OCH

    def obody(k, carry):
        r = s * RPT + k * OCH
        ia = pltpu.async_copy(acc.at[pl.ds(r, OCH)], stg, isemA)
        if scaled:
            ib = pltpu.async_copy(sc16.at[pl.ds(r, OCH)], scb, isemB)
        ia.wait()
        if scaled:
            ib.wait()
            for i in range(OCH):
                v = scb[i, :]
                for k2 in range(DH // 16):
                    stg[i, pl.ds(k2 * 16, 16)] = (
                        stg[i, pl.ds(k2 * 16, 16)] * v)
        if concat_out:
            pltpu.sync_copy(stg, out.at[pl.ds(r, OCH), pl.ds(c * DH, DH)])
        else:
            pltpu.sync_copy(stg, out.at[pl.ds(c * N + r, OCH)])
        return carry
    lax.fori_loop(0, nch, obody, None)


_SEG_SCRATCH = ([pltpu.VMEM((BODY,), jnp.int32),
                 pltpu.VMEM((BODY,), jnp.int32),
                 pltpu.VMEM((K, CHUNK), jnp.int32),
                 pltpu.VMEM((K, CHUNK), jnp.int32)]
                + [pltpu.VMEM((CHUNK, DH), _f32)] * 5
                + [pltpu.VMEM((OCH, DH), _f32)])
_SEG_SEMS = [pltpu.SemaphoreType.DMA] * 8


@functools.partial(
    pl.kernel,
    mesh=_mesh,
    out_type=jax.ShapeDtypeStruct((2 * N, DH), _f32),
    scratch_types=(_SEG_SCRATCH
                   + [pltpu.VMEM_SHARED((NPAD, DH), _f32)]
                   + _SEG_SEMS),
)
def _sc_segsum(src2, dst3, tab, out,
               sidxA, sidxB, didxA, didxB,
               rows0, rows1, rows2, rows3, rows4, stg, acc,
               g0, g1, g2, g3, g4, ssem, isemA, isemB):
    _seg_impl(src2, dst3, tab, None, out,
              sidxA, sidxB, didxA, didxB,
              rows0, rows1, rows2, rows3, rows4, stg, None, acc,
              g0, g1, g2, g3, g4, ssem, isemA, isemB, scaled=False)


@functools.partial(
    pl.kernel,
    mesh=_mesh,
    out_type=jax.ShapeDtypeStruct((2 * N, DH), _f32),
    scratch_types=(_SEG_SCRATCH
                   + [pltpu.VMEM((OCH, DEGW), _f32)]
                   + [pltpu.VMEM_SHARED((NPAD, DH), _f32)]
                   + _SEG_SEMS),
)
def _sc_segsum_scaled(src2, dst3, tab, sc16, out,
                      sidxA, sidxB, didxA, didxB,
                      rows0, rows1, rows2, rows3, rows4, stg, scb, acc,
                      g0, g1, g2, g3, g4, ssem, isemA, isemB):
    _seg_impl(src2, dst3, tab, sc16, out,
              sidxA, sidxB, didxA, didxB,
              rows0, rows1, rows2, rows3, rows4, stg, scb, acc,
              g0, g1, g2, g3, g4, ssem, isemA, isemB, scaled=True)


@functools.partial(
    pl.kernel,
    mesh=_mesh,
    out_type=jax.ShapeDtypeStruct((N, D), _f32),
    scratch_types=(_SEG_SCRATCH
                   + [pltpu.VMEM((OCH, DEGW), _f32)]
                   + [pltpu.VMEM_SHARED((NPAD, DH), _f32)]
                   + _SEG_SEMS),
)
def _sc_segsum_final(src2, dst3, tab, sc16, out,
                     sidxA, sidxB, didxA, didxB,
                     rows0, rows1, rows2, rows3, rows4, stg, scb, acc,
                     g0, g1, g2, g3, g4, ssem, isemA, isemB):
    _seg_impl(src2, dst3, tab, sc16, out,
              sidxA, sidxB, didxA, didxB,
              rows0, rows1, rows2, rows3, rows4, stg, scb, acc,
              g0, g1, g2, g3, g4, ssem, isemA, isemB, scaled=True,
              concat_out=True)


def _mm1_body(x, w, b, y):
    yy = lax.dot_general(x[...], w[...], (((1,), (1,)), ((), ())),
                         preferred_element_type=_f32)
    y[...] = yy + b[...]


def _tc_mm1(x, w1, b1r):
    return pl.pallas_call(
        _mm1_body,
        grid=(GRID, 2),
        in_specs=[pl.BlockSpec((BM, D), lambda i, j: (i, 0)),
                  pl.BlockSpec((DH, D), lambda i, j: (j, 0)),
                  pl.BlockSpec((1, DH), lambda i, j: (0, j))],
        out_specs=pl.BlockSpec((BM, DH), lambda i, j: (j * GRID + i, 0)),
        out_shape=jax.ShapeDtypeStruct((2 * N, DH), _f32),
    )(x, w1, b1r)


def _prep_body(dvd, ded, y1p, dv, de, y1):
    d = lax.rsqrt(dvd[...])
    dv[...] = d
    de[...] = 1.0 / ded[...]
    y1[...] = y1p[...] * d[:, :1]


def _tc_prep(deg, y1p):
    return pl.pallas_call(
        _prep_body,
        grid=(GRID, 2),
        in_specs=[pl.BlockSpec((BM, DEGW), lambda i, j: (i, 0)),
                  pl.BlockSpec((BM, DEGW), lambda i, j: (GRID + i, 0)),
                  pl.BlockSpec((BM, DH), lambda i, j: (j * GRID + i, 0))],
        out_specs=[pl.BlockSpec((BM, DEGW), lambda i, j: (i, 0)),
                   pl.BlockSpec((BM, DEGW), lambda i, j: (i, 0)),
                   pl.BlockSpec((BM, DH), lambda i, j: (j * GRID + i, 0))],
        out_shape=[jax.ShapeDtypeStruct((N, DEGW), _f32),
                   jax.ShapeDtypeStruct((N, DEGW), _f32),
                   jax.ShapeDtypeStruct((2 * N, DH), _f32)],
    )(deg, deg, y1p)


def _mid_body(za, zbr, dv, w, b, y):
    d = dv[...][:, :1]
    h = jnp.concatenate([jnp.maximum(za[...] * d, 0.0),
                         jnp.maximum(zbr[...] * d, 0.0)], axis=1)
    yy = lax.dot_general(h, w[...], (((1,), (1,)), ((), ())),
                         preferred_element_type=_f32)
    y[...] = (yy + b[...]) * d


def _tc_mid(zv, dv, w2, b2r):
    return pl.pallas_call(
        _mid_body,
        grid=(GRID, 2),
        in_specs=[pl.BlockSpec((BM, DH), lambda i, j: (i, 0)),
                  pl.BlockSpec((BM, DH), lambda i, j: (GRID + i, 0)),
                  pl.BlockSpec((BM, DEGW), lambda i, j: (i, 0)),
                  pl.BlockSpec((DH, D), lambda i, j: (j, 0)),
                  pl.BlockSpec((1, DH), lambda i, j: (0, j))],
        out_specs=pl.BlockSpec((BM, DH), lambda i, j: (j * GRID + i, 0)),
        out_shape=jax.ShapeDtypeStruct((2 * N, DH), _f32),
    )(zv, zv, dv, w2, b2r)


def _final_body(za, zbr, dv, o):
    d = dv[...][:, :1]
    o[...] = jnp.concatenate([za[...] * d, zbr[...] * d], axis=1)


def _tc_final(zv, dv):
    return pl.pallas_call(
        _final_body,
        grid=(GRID,),
        in_specs=[pl.BlockSpec((BM, DH), lambda i: (i, 0)),
                  pl.BlockSpec((BM, DH), lambda i: (GRID + i, 0)),
                  pl.BlockSpec((BM, DEGW), lambda i: (i, 0))],
        out_specs=pl.BlockSpec((BM, D), lambda i: (i, 0)),
        out_shape=jax.ShapeDtypeStruct((N, D), _f32),
    )(zv, zv, dv)


def kernel(X, W1, b1, W2, b2, node_idx, edge_idx):
    b1r = b1.reshape(1, D)
    b2r = b2.reshape(1, D)
    cidx3 = jnp.concatenate([node_idx, edge_idx]).reshape(DROW3, DK, DCH)
    s_node = jnp.concatenate([node_idx, node_idx + N])
    s_edge = jnp.concatenate([edge_idx, edge_idx + N])
    d_node = node_idx.reshape(NROW3, K, CHUNK)
    d_edge = edge_idx.reshape(NROW3, K, CHUNK)
    y1p = _tc_mm1(X, W1, b1r)
    deg = _sc_degrees(cidx3)
    dv, de, y1 = _tc_prep(deg, y1p)
    ze = _sc_segsum_scaled(s_node, d_edge, y1, de)
    zv = _sc_segsum(s_edge, d_node, ze)
    y2 = _tc_mid(zv, dv, W2, b2r)
    z2 = _sc_segsum_scaled(s_node, d_edge, y2, de)
    return _sc_segsum_final(s_edge, d_node, z2, dv)


# speedup vs baseline: 7.3361x; 1.0105x over previous
import functools

import jax
import jax.numpy as jnp
from jax import lax
from jax.experimental import pallas as pl
from jax.experimental.pallas import tpu as pltpu
from jax.experimental.pallas import tpu_sc as plsc

N = 10000
NNZ = 160000
D = 256
DH = 128
NS = 16
PER_TILE = NNZ // NS
CHUNK = 40
K = 5
BODY = K * CHUNK
NBODY = PER_TILE // BODY
NROW3 = NNZ // BODY
NPAD = 10240
RPT = NPAD // NS
OCH = 80
DEGW = 16
DCH = 125
DK = 4
DNB = PER_TILE // (DK * DCH)
DROW3 = 2 * NNZ // (DK * DCH)
BM = 1000
GRID = N // BM

_f32 = jnp.float32
_mesh = plsc.VectorSubcoreMesh(core_axis_name="c", subcore_axis_name="s")


@functools.partial(
    pl.kernel,
    mesh=_mesh,
    out_type=jax.ShapeDtypeStruct((2 * N, DEGW), _f32),
    scratch_types=[pltpu.VMEM((DK, DCH), jnp.int32),
                   pltpu.VMEM((DK, DCH), jnp.int32),
                   pltpu.VMEM((DCH, DEGW), _f32),
                   pltpu.VMEM((OCH, DEGW), _f32),
                   pltpu.VMEM((OCH, DEGW), _f32),
                   pltpu.VMEM_SHARED((NPAD, DEGW), _f32),
                   pltpu.SemaphoreType.DMA,
                   pltpu.SemaphoreType.DMA,
                   pltpu.SemaphoreType.DMA],
)
def _sc_degrees(cidx3, deg_out, didxA, didxB, onesb, zb, ob, acc,
                ssem, isemA, isemB):
    c = lax.axis_index("c")
    s = lax.axis_index("s")
    ones16 = jnp.ones((16,), _f32)
    zero16 = jnp.zeros((16,), _f32)
    for i in range(DCH):
        onesb[i, :] = ones16
    for i in range(OCH):
        zb[i, :] = zero16

    zcs = [pltpu.async_copy(zb, acc.at[pl.ds(s * RPT + k * OCH, OCH)],
                            isemA) for k in range(RPT // OCH)]
    for zc in zcs:
        zc.wait()
    plsc.subcore_barrier()

    rb = c * (DROW3 // 2) + s * DNB
    pltpu.sync_copy(cidx3.at[rb], didxA)

    def body(j, carry):
        rA2 = rb + jnp.minimum(2 * j + 2, DNB - 1)
        sA = [pltpu.async_copy(onesb, acc.at[didxA.at[i]], ssem, add=True)
              for i in range(DK)]
        iB = pltpu.async_copy(cidx3.at[rb + 2 * j + 1], didxB, isemB)
        for i in range(DK):
            sA[i].wait()
        iA = pltpu.async_copy(cidx3.at[rA2], didxA, isemA)
        iB.wait()
        sB = [pltpu.async_copy(onesb, acc.at[didxB.at[i]], ssem, add=True)
              for i in range(DK)]
        for i in range(DK):
            sB[i].wait()
        iA.wait()
        return carry
    lax.fori_loop(0, DNB // 2, body, None)
    plsc.subcore_barrier()

    nch = jnp.minimum(jnp.maximum(N - s * RPT, 0), RPT) // OCH

    def obody(k, carry):
        r = s * RPT + k * OCH
        pltpu.sync_copy(acc.at[pl.ds(r, OCH)], ob)
        pltpu.sync_copy(ob, deg_out.at[pl.ds(c * N + r, OCH)])
        return carry
    lax.fori_loop(0, nch, obody, None)


def _seg_impl(src2, dst3, tab, sc16, out,
              sidxA, sidxB, didxA, didxB,
              rows0, rows1, rows2, rows3, rows4, stg, scb, acc,
              g0, g1, g2, g3, g4, ssem, isemA, isemB, scaled,
              concat_out=False):
    c = lax.axis_index("c")
    s = lax.axis_index("s")
    rows = (rows0, rows1, rows2, rows3, rows4)
    gsems = (g0, g1, g2, g3, g4)
    zero16 = jnp.zeros((16,), _f32)
    for i in range(OCH):
        for k in range(DH // 16):
            stg[i, pl.ds(k * 16, 16)] = zero16

    zcs = [pltpu.async_copy(stg, acc.at[pl.ds(s * RPT + k * OCH, OCH)],
                            isemA) for k in range(RPT // OCH)]
    for zc in zcs:
        zc.wait()
    plsc.subcore_barrier()

    sbase = c * NNZ + s * PER_TILE
    rbase = s * NBODY

    pltpu.sync_copy(src2.at[pl.ds(sbase, BODY)], sidxA)
    pltpu.sync_copy(dst3.at[rbase], didxA)

    def body(j, carry):
        jA, jB, jA2 = 2 * j, 2 * j + 1, 2 * j + 2
        jA2c = jnp.minimum(jA2, NBODY - 1)

        gA = [pltpu.async_copy(tab.at[sidxA.at[pl.ds(i * CHUNK, CHUNK)]],
                               rows[i], gsems[i]) for i in range(K)]
        iB0 = pltpu.async_copy(src2.at[pl.ds(sbase + jB * BODY, BODY)],
                               sidxB, isemB)
        iB1 = pltpu.async_copy(dst3.at[rbase + jB], didxB, isemB)
        sA = []
        for i in range(K):
            gA[i].wait()
            sA.append(pltpu.async_copy(rows[i], acc.at[didxA.at[i]], ssem,
                                       add=True))
        iA0 = pltpu.async_copy(src2.at[pl.ds(sbase + jA2c * BODY, BODY)],
                               sidxA, isemA)
        iB0.wait()
        iB1.wait()
        gB = []
        for i in range(K):
            sA[i].wait()
            gB.append(pltpu.async_copy(tab.at[sidxB.at[pl.ds(i * CHUNK,
                                                             CHUNK)]],
                                       rows[i], gsems[i]))
        iA1 = pltpu.async_copy(dst3.at[rbase + jA2c], didxA, isemA)
        sB = []
        for i in range(K):
            gB[i].wait()
            sB.append(pltpu.async_copy(rows[i], acc.at[didxB.at[i]], ssem,
                                       add=True))
        for i in range(K):
            sB[i].wait()
        iA0.wait()
        iA1.wait()
        return carry
    lax.fori_loop(0, NBODY // 2, body, None)
    plsc.subcore_barrier()

    nch = jnp.minimum(jnp.maximum(N - s * RPT, 0), RPT) // OCH

    def _oslice(r):
        if concat_out:
            return out.at[pl.ds(r, CHUNK), pl.ds(c * DH, DH)]
        return out.at[pl.ds(c * N + r, CHUNK)]

    def _oscale(buf, soff):
        for i in range(CHUNK):
            v = scb[soff + i, :]
            for k2 in range(DH // 16):
                buf[i, pl.ds(k2 * 16, 16)] = buf[i, pl.ds(k2 * 16, 16)] * v

    def obody(k, carry):
        r0 = s * RPT + (2 * k) * CHUNK
        r1 = r0 + CHUNK
        ia = pltpu.async_copy(acc.at[pl.ds(r0, CHUNK)], rows0, g0)
        ib = pltpu.async_copy(acc.at[pl.ds(r1, CHUNK)], rows1, g1)
        if scaled:
            sa = pltpu.async_copy(sc16.at[pl.ds(r0, CHUNK)],
                                  scb.at[pl.ds(0, CHUNK)], g2)
            sb = pltpu.async_copy(sc16.at[pl.ds(r1, CHUNK)],
                                  scb.at[pl.ds(CHUNK, CHUNK)], g3)
        ia.wait()
        if scaled:
            sa.wait()
            _oscale(rows0, 0)
        oa = pltpu.async_copy(rows0, _oslice(r0), g4)
        ib.wait()
        if scaled:
            sb.wait()
            _oscale(rows1, CHUNK)
        ob = pltpu.async_copy(rows1, _oslice(r1), ssem)
        oa.wait()
        ob.wait()
        return carry
    lax.fori_loop(0, nch * OCH // (2 * CHUNK), obody, None)


_SEG_SCRATCH = ([pltpu.VMEM((BODY,), jnp.int32),
                 pltpu.VMEM((BODY,), jnp.int32),
                 pltpu.VMEM((K, CHUNK), jnp.int32),
                 pltpu.VMEM((K, CHUNK), jnp.int32)]
                + [pltpu.VMEM((CHUNK, DH), _f32)] * 5
                + [pltpu.VMEM((OCH, DH), _f32)])
_SEG_SEMS = [pltpu.SemaphoreType.DMA] * 8


@functools.partial(
    pl.kernel,
    mesh=_mesh,
    out_type=jax.ShapeDtypeStruct((2 * N, DH), _f32),
    scratch_types=(_SEG_SCRATCH
                   + [pltpu.VMEM_SHARED((NPAD, DH), _f32)]
                   + _SEG_SEMS),
)
def _sc_segsum(src2, dst3, tab, out,
               sidxA, sidxB, didxA, didxB,
               rows0, rows1, rows2, rows3, rows4, stg, acc,
               g0, g1, g2, g3, g4, ssem, isemA, isemB):
    _seg_impl(src2, dst3, tab, None, out,
              sidxA, sidxB, didxA, didxB,
              rows0, rows1, rows2, rows3, rows4, stg, None, acc,
              g0, g1, g2, g3, g4, ssem, isemA, isemB, scaled=False)


@functools.partial(
    pl.kernel,
    mesh=_mesh,
    out_type=jax.ShapeDtypeStruct((2 * N, DH), _f32),
    scratch_types=(_SEG_SCRATCH
                   + [pltpu.VMEM((OCH, DEGW), _f32)]
                   + [pltpu.VMEM_SHARED((NPAD, DH), _f32)]
                   + _SEG_SEMS),
)
def _sc_segsum_scaled(src2, dst3, tab, sc16, out,
                      sidxA, sidxB, didxA, didxB,
                      rows0, rows1, rows2, rows3, rows4, stg, scb, acc,
                      g0, g1, g2, g3, g4, ssem, isemA, isemB):
    _seg_impl(src2, dst3, tab, sc16, out,
              sidxA, sidxB, didxA, didxB,
              rows0, rows1, rows2, rows3, rows4, stg, scb, acc,
              g0, g1, g2, g3, g4, ssem, isemA, isemB, scaled=True)


@functools.partial(
    pl.kernel,
    mesh=_mesh,
    out_type=jax.ShapeDtypeStruct((N, D), _f32),
    scratch_types=(_SEG_SCRATCH
                   + [pltpu.VMEM((OCH, DEGW), _f32)]
                   + [pltpu.VMEM_SHARED((NPAD, DH), _f32)]
                   + _SEG_SEMS),
)
def _sc_segsum_final(src2, dst3, tab, sc16, out,
                     sidxA, sidxB, didxA, didxB,
                     rows0, rows1, rows2, rows3, rows4, stg, scb, acc,
                     g0, g1, g2, g3, g4, ssem, isemA, isemB):
    _seg_impl(src2, dst3, tab, sc16, out,
              sidxA, sidxB, didxA, didxB,
              rows0, rows1, rows2, rows3, rows4, stg, scb, acc,
              g0, g1, g2, g3, g4, ssem, isemA, isemB, scaled=True,
              concat_out=True)


def _mm1_body(x, w, b, y):
    yy = lax.dot_general(x[...], w[...], (((1,), (1,)), ((), ())),
                         preferred_element_type=_f32)
    y[...] = yy + b[...]


def _tc_mm1(x, w1, b1r):
    return pl.pallas_call(
        _mm1_body,
        grid=(GRID, 2),
        in_specs=[pl.BlockSpec((BM, D), lambda i, j: (i, 0)),
                  pl.BlockSpec((DH, D), lambda i, j: (j, 0)),
                  pl.BlockSpec((1, DH), lambda i, j: (0, j))],
        out_specs=pl.BlockSpec((BM, DH), lambda i, j: (j * GRID + i, 0)),
        out_shape=jax.ShapeDtypeStruct((2 * N, DH), _f32),
    )(x, w1, b1r)


def _prep_body(dvd, ded, y1p, dv, de, y1):
    d = lax.rsqrt(dvd[...])
    dv[...] = d
    de[...] = 1.0 / ded[...]
    y1[...] = y1p[...] * d[:, :1]


def _tc_prep(deg, y1p):
    return pl.pallas_call(
        _prep_body,
        grid=(GRID, 2),
        in_specs=[pl.BlockSpec((BM, DEGW), lambda i, j: (i, 0)),
                  pl.BlockSpec((BM, DEGW), lambda i, j: (GRID + i, 0)),
                  pl.BlockSpec((BM, DH), lambda i, j: (j * GRID + i, 0))],
        out_specs=[pl.BlockSpec((BM, DEGW), lambda i, j: (i, 0)),
                   pl.BlockSpec((BM, DEGW), lambda i, j: (i, 0)),
                   pl.BlockSpec((BM, DH), lambda i, j: (j * GRID + i, 0))],
        out_shape=[jax.ShapeDtypeStruct((N, DEGW), _f32),
                   jax.ShapeDtypeStruct((N, DEGW), _f32),
                   jax.ShapeDtypeStruct((2 * N, DH), _f32)],
    )(deg, deg, y1p)


def _mid_body(za, zbr, dv, w, b, y):
    d = dv[...][:, :1]
    h = jnp.concatenate([jnp.maximum(za[...] * d, 0.0),
                         jnp.maximum(zbr[...] * d, 0.0)], axis=1)
    yy = lax.dot_general(h, w[...], (((1,), (1,)), ((), ())),
                         preferred_element_type=_f32)
    y[...] = (yy + b[...]) * d


def _tc_mid(zv, dv, w2, b2r):
    return pl.pallas_call(
        _mid_body,
        grid=(GRID, 2),
        in_specs=[pl.BlockSpec((BM, DH), lambda i, j: (i, 0)),
                  pl.BlockSpec((BM, DH), lambda i, j: (GRID + i, 0)),
                  pl.BlockSpec((BM, DEGW), lambda i, j: (i, 0)),
                  pl.BlockSpec((DH, D), lambda i, j: (j, 0)),
                  pl.BlockSpec((1, DH), lambda i, j: (0, j))],
        out_specs=pl.BlockSpec((BM, DH), lambda i, j: (j * GRID + i, 0)),
        out_shape=jax.ShapeDtypeStruct((2 * N, DH), _f32),
    )(zv, zv, dv, w2, b2r)


def _final_body(za, zbr, dv, o):
    d = dv[...][:, :1]
    o[...] = jnp.concatenate([za[...] * d, zbr[...] * d], axis=1)


def _tc_final(zv, dv):
    return pl.pallas_call(
        _final_body,
        grid=(GRID,),
        in_specs=[pl.BlockSpec((BM, DH), lambda i: (i, 0)),
                  pl.BlockSpec((BM, DH), lambda i: (GRID + i, 0)),
                  pl.BlockSpec((BM, DEGW), lambda i: (i, 0))],
        out_specs=pl.BlockSpec((BM, D), lambda i: (i, 0)),
        out_shape=jax.ShapeDtypeStruct((N, D), _f32),
    )(zv, zv, dv)


def kernel(X, W1, b1, W2, b2, node_idx, edge_idx):
    b1r = b1.reshape(1, D)
    b2r = b2.reshape(1, D)
    cidx3 = jnp.concatenate([node_idx, edge_idx]).reshape(DROW3, DK, DCH)
    s_node = jnp.concatenate([node_idx, node_idx + N])
    s_edge = jnp.concatenate([edge_idx, edge_idx + N])
    d_node = node_idx.reshape(NROW3, K, CHUNK)
    d_edge = edge_idx.reshape(NROW3, K, CHUNK)
    y1p = _tc_mm1(X, W1, b1r)
    deg = _sc_degrees(cidx3)
    dv, de, y1 = _tc_prep(deg, y1p)
    ze = _sc_segsum_scaled(s_node, d_edge, y1, de)
    zv = _sc_segsum(s_edge, d_node, ze)
    y2 = _tc_mid(zv, dv, W2, b2r)
    z2 = _sc_segsum_scaled(s_node, d_edge, y2, de)
    return _sc_segsum_final(s_edge, d_node, z2, dv)


# speedup vs baseline: 7.3651x; 1.0040x over previous
import functools

import jax
import jax.numpy as jnp
from jax import lax
from jax.experimental import pallas as pl
from jax.experimental.pallas import tpu as pltpu
from jax.experimental.pallas import tpu_sc as plsc

N = 10000
NNZ = 160000
D = 256
DH = 128
NS = 16
PER_TILE = NNZ // NS
CHUNK = 40
K = 5
BODY = K * CHUNK
NBODY = PER_TILE // BODY
NROW3 = NNZ // BODY
NPAD = 10240
RPT = NPAD // NS
OCH = 80
DEGW = 16
DCH = 125
DK = 4
DNB = PER_TILE // (DK * DCH)
DROW3 = 2 * NNZ // (DK * DCH)
BM = 1000
GRID = N // BM

_f32 = jnp.float32
_mesh = plsc.VectorSubcoreMesh(core_axis_name="c", subcore_axis_name="s")


@functools.partial(
    pl.kernel,
    mesh=_mesh,
    out_type=jax.ShapeDtypeStruct((2 * N, DEGW), _f32),
    scratch_types=[pltpu.VMEM((DK, DCH), jnp.int32),
                   pltpu.VMEM((DK, DCH), jnp.int32),
                   pltpu.VMEM((DCH, DEGW), _f32),
                   pltpu.VMEM((OCH, DEGW), _f32),
                   pltpu.VMEM((OCH, DEGW), _f32),
                   pltpu.VMEM_SHARED((NPAD, DEGW), _f32),
                   pltpu.SemaphoreType.DMA,
                   pltpu.SemaphoreType.DMA,
                   pltpu.SemaphoreType.DMA],
)
def _sc_degrees(cidx3, deg_out, didxA, didxB, onesb, zb, ob, acc,
                ssem, isemA, isemB):
    c = lax.axis_index("c")
    s = lax.axis_index("s")
    ones16 = jnp.ones((16,), _f32)
    zero16 = jnp.zeros((16,), _f32)
    for i in range(DCH):
        onesb[i, :] = ones16
    for i in range(OCH):
        zb[i, :] = zero16

    zcs = [pltpu.async_copy(zb, acc.at[pl.ds(s * RPT + k * OCH, OCH)],
                            isemA) for k in range(RPT // OCH)]
    for zc in zcs:
        zc.wait()
    plsc.subcore_barrier()

    rb = c * (DROW3 // 2) + s * DNB
    pltpu.sync_copy(cidx3.at[rb], didxA)

    def body(j, carry):
        rA2 = rb + jnp.minimum(2 * j + 2, DNB - 1)
        sA = [pltpu.async_copy(onesb, acc.at[didxA.at[i]], ssem, add=True)
              for i in range(DK)]
        iB = pltpu.async_copy(cidx3.at[rb + 2 * j + 1], didxB, isemB)
        for i in range(DK):
            sA[i].wait()
        iA = pltpu.async_copy(cidx3.at[rA2], didxA, isemA)
        iB.wait()
        sB = [pltpu.async_copy(onesb, acc.at[didxB.at[i]], ssem, add=True)
              for i in range(DK)]
        for i in range(DK):
            sB[i].wait()
        iA.wait()
        return carry
    lax.fori_loop(0, DNB // 2, body, None)
    plsc.subcore_barrier()

    nch = jnp.minimum(jnp.maximum(N - s * RPT, 0), RPT) // OCH

    def obody(k, carry):
        r = s * RPT + k * OCH
        pltpu.sync_copy(acc.at[pl.ds(r, OCH)], ob)
        pltpu.sync_copy(ob, deg_out.at[pl.ds(c * N + r, OCH)])
        return carry
    lax.fori_loop(0, nch, obody, None)


def _seg_impl(src2, dst3, tab, sc16, out,
              sidxA, sidxB, didxA, didxB,
              rows0, rows1, rows2, rows3, rows4, stg, scb, acc,
              g0, g1, g2, g3, g4, ssem, isemA, isemB, scaled,
              concat_out=False):
    c = lax.axis_index("c")
    s = lax.axis_index("s")
    rows = (rows0, rows1, rows2, rows3, rows4)
    gsems = (g0, g1, g2, g3, g4)
    zero16 = jnp.zeros((16,), _f32)
    for i in range(OCH):
        for k in range(DH // 16):
            stg[i, pl.ds(k * 16, 16)] = zero16

    zcs = [pltpu.async_copy(stg, acc.at[pl.ds(s * RPT + k * OCH, OCH)],
                            isemA) for k in range(RPT // OCH)]
    for zc in zcs:
        zc.wait()
    plsc.subcore_barrier()

    sbase = c * NNZ + s * PER_TILE
    rbase = s * NBODY

    pltpu.sync_copy(src2.at[pl.ds(sbase, BODY)], sidxA)
    pltpu.sync_copy(dst3.at[rbase], didxA)

    def body(j, carry):
        jA, jB, jA2 = 2 * j, 2 * j + 1, 2 * j + 2
        jA2c = jnp.minimum(jA2, NBODY - 1)

        gA = [pltpu.async_copy(tab.at[sidxA.at[pl.ds(i * CHUNK, CHUNK)]],
                               rows[i], gsems[i]) for i in range(K)]
        iB0 = pltpu.async_copy(src2.at[pl.ds(sbase + jB * BODY, BODY)],
                               sidxB, isemB)
        iB1 = pltpu.async_copy(dst3.at[rbase + jB], didxB, isemB)
        sA = []
        for i in range(K):
            gA[i].wait()
            sA.append(pltpu.async_copy(rows[i], acc.at[didxA.at[i]], ssem,
                                       add=True))
        iA0 = pltpu.async_copy(src2.at[pl.ds(sbase + jA2c * BODY, BODY)],
                               sidxA, isemA)
        iB0.wait()
        iB1.wait()
        gB = []
        for i in range(K):
            sA[i].wait()
            gB.append(pltpu.async_copy(tab.at[sidxB.at[pl.ds(i * CHUNK,
                                                             CHUNK)]],
                                       rows[i], gsems[i]))
        iA1 = pltpu.async_copy(dst3.at[rbase + jA2c], didxA, isemA)
        sB = []
        for i in range(K):
            gB[i].wait()
            sB.append(pltpu.async_copy(rows[i], acc.at[didxB.at[i]], ssem,
                                       add=True))
        for i in range(K):
            sB[i].wait()
        iA0.wait()
        iA1.wait()
        return carry
    lax.fori_loop(0, NBODY // 2, body, None)
    plsc.subcore_barrier()

    nch = jnp.minimum(jnp.maximum(N - s * RPT, 0), RPT) // OCH

    def _oslice(r):
        if concat_out:
            return out.at[pl.ds(r, CHUNK), pl.ds(c * DH, DH)]
        return out.at[pl.ds(c * N + r, CHUNK)]

    def _oscale(buf, soff):
        for i in range(CHUNK):
            v = scb[soff + i, :]
            for k2 in range(DH // 16):
                buf[i, pl.ds(k2 * 16, 16)] = buf[i, pl.ds(k2 * 16, 16)] * v

    def obody(k, carry):
        r0 = s * RPT + (2 * k) * CHUNK
        r1 = r0 + CHUNK
        ia = pltpu.async_copy(acc.at[pl.ds(r0, CHUNK)], rows0, g0)
        ib = pltpu.async_copy(acc.at[pl.ds(r1, CHUNK)], rows1, g1)
        if scaled:
            sa = pltpu.async_copy(sc16.at[pl.ds(r0, CHUNK)],
                                  scb.at[pl.ds(0, CHUNK)], g2)
            sb = pltpu.async_copy(sc16.at[pl.ds(r1, CHUNK)],
                                  scb.at[pl.ds(CHUNK, CHUNK)], g3)
        ia.wait()
        if scaled:
            sa.wait()
            _oscale(rows0, 0)
        oa = pltpu.async_copy(rows0, _oslice(r0), g4)
        ib.wait()
        if scaled:
            sb.wait()
            _oscale(rows1, CHUNK)
        ob = pltpu.async_copy(rows1, _oslice(r1), ssem)
        oa.wait()
        ob.wait()
        return carry
    lax.fori_loop(0, nch * OCH // (2 * CHUNK), obody, None)


_SEG_SCRATCH = ([pltpu.VMEM((BODY,), jnp.int32),
                 pltpu.VMEM((BODY,), jnp.int32),
                 pltpu.VMEM((K, CHUNK), jnp.int32),
                 pltpu.VMEM((K, CHUNK), jnp.int32)]
                + [pltpu.VMEM((CHUNK, DH), _f32)] * 5
                + [pltpu.VMEM((OCH, DH), _f32)])
_SEG_SEMS = [pltpu.SemaphoreType.DMA] * 8


@functools.partial(
    pl.kernel,
    mesh=_mesh,
    out_type=jax.ShapeDtypeStruct((2 * N, DH), _f32),
    scratch_types=(_SEG_SCRATCH
                   + [pltpu.VMEM_SHARED((NPAD, DH), _f32)]
                   + _SEG_SEMS),
)
def _sc_segsum(src2, dst3, tab, out,
               sidxA, sidxB, didxA, didxB,
               rows0, rows1, rows2, rows3, rows4, stg, acc,
               g0, g1, g2, g3, g4, ssem, isemA, isemB):
    _seg_impl(src2, dst3, tab, None, out,
              sidxA, sidxB, didxA, didxB,
              rows0, rows1, rows2, rows3, rows4, stg, None, acc,
              g0, g1, g2, g3, g4, ssem, isemA, isemB, scaled=False)


@functools.partial(
    pl.kernel,
    mesh=_mesh,
    out_type=jax.ShapeDtypeStruct((2 * N, DH), _f32),
    scratch_types=(_SEG_SCRATCH
                   + [pltpu.VMEM((OCH, DEGW), _f32)]
                   + [pltpu.VMEM_SHARED((NPAD, DH), _f32)]
                   + _SEG_SEMS),
)
def _sc_segsum_scaled(src2, dst3, tab, sc16, out,
                      sidxA, sidxB, didxA, didxB,
                      rows0, rows1, rows2, rows3, rows4, stg, scb, acc,
                      g0, g1, g2, g3, g4, ssem, isemA, isemB):
    _seg_impl(src2, dst3, tab, sc16, out,
              sidxA, sidxB, didxA, didxB,
              rows0, rows1, rows2, rows3, rows4, stg, scb, acc,
              g0, g1, g2, g3, g4, ssem, isemA, isemB, scaled=True)


@functools.partial(
    pl.kernel,
    mesh=_mesh,
    out_type=jax.ShapeDtypeStruct((N, D), _f32),
    scratch_types=(_SEG_SCRATCH
                   + [pltpu.VMEM((OCH, DEGW), _f32)]
                   + [pltpu.VMEM_SHARED((NPAD, DH), _f32)]
                   + _SEG_SEMS),
)
def _sc_segsum_final(src2, dst3, tab, sc16, out,
                     sidxA, sidxB, didxA, didxB,
                     rows0, rows1, rows2, rows3, rows4, stg, scb, acc,
                     g0, g1, g2, g3, g4, ssem, isemA, isemB):
    _seg_impl(src2, dst3, tab, sc16, out,
              sidxA, sidxB, didxA, didxB,
              rows0, rows1, rows2, rows3, rows4, stg, scb, acc,
              g0, g1, g2, g3, g4, ssem, isemA, isemB, scaled=True,
              concat_out=True)


def _mm1_body(x, w, b, y):
    yy = lax.dot_general(x[...], w[...], (((1,), (1,)), ((), ())),
                         preferred_element_type=_f32)
    y[...] = yy + b[...]


def _tc_mm1(x, w1, b1r):
    return pl.pallas_call(
        _mm1_body,
        grid=(GRID, 2),
        in_specs=[pl.BlockSpec((BM, D), lambda i, j: (i, 0)),
                  pl.BlockSpec((DH, D), lambda i, j: (j, 0)),
                  pl.BlockSpec((1, DH), lambda i, j: (0, j))],
        out_specs=pl.BlockSpec((BM, DH), lambda i, j: (j * GRID + i, 0)),
        out_shape=jax.ShapeDtypeStruct((2 * N, DH), _f32),
    )(x, w1, b1r)


def _mm1p_body(x, w, b, dvd, ded, dv, de, y1):
    d = lax.rsqrt(dvd[...])
    dv[...] = d
    de[...] = 1.0 / ded[...]
    yy = lax.dot_general(x[...], w[...], (((1,), (1,)), ((), ())),
                         preferred_element_type=_f32)
    y1[...] = (yy + b[...]) * d[:, :1]


def _tc_mm1_prep(x, w1, b1r, deg):
    return pl.pallas_call(
        _mm1p_body,
        grid=(GRID, 2),
        in_specs=[pl.BlockSpec((BM, D), lambda i, j: (i, 0)),
                  pl.BlockSpec((DH, D), lambda i, j: (j, 0)),
                  pl.BlockSpec((1, DH), lambda i, j: (0, j)),
                  pl.BlockSpec((BM, DEGW), lambda i, j: (i, 0)),
                  pl.BlockSpec((BM, DEGW), lambda i, j: (GRID + i, 0))],
        out_specs=[pl.BlockSpec((BM, DEGW), lambda i, j: (i, 0)),
                   pl.BlockSpec((BM, DEGW), lambda i, j: (i, 0)),
                   pl.BlockSpec((BM, DH), lambda i, j: (j * GRID + i, 0))],
        out_shape=[jax.ShapeDtypeStruct((N, DEGW), _f32),
                   jax.ShapeDtypeStruct((N, DEGW), _f32),
                   jax.ShapeDtypeStruct((2 * N, DH), _f32)],
    )(x, w1, b1r, deg, deg)


def _mid_body(za, zbr, dv, w, b, y):
    d = dv[...][:, :1]
    h = jnp.concatenate([jnp.maximum(za[...] * d, 0.0),
                         jnp.maximum(zbr[...] * d, 0.0)], axis=1)
    yy = lax.dot_general(h, w[...], (((1,), (1,)), ((), ())),
                         preferred_element_type=_f32)
    y[...] = (yy + b[...]) * d


def _tc_mid(zv, dv, w2, b2r):
    return pl.pallas_call(
        _mid_body,
        grid=(GRID, 2),
        in_specs=[pl.BlockSpec((BM, DH), lambda i, j: (i, 0)),
                  pl.BlockSpec((BM, DH), lambda i, j: (GRID + i, 0)),
                  pl.BlockSpec((BM, DEGW), lambda i, j: (i, 0)),
                  pl.BlockSpec((DH, D), lambda i, j: (j, 0)),
                  pl.BlockSpec((1, DH), lambda i, j: (0, j))],
        out_specs=pl.BlockSpec((BM, DH), lambda i, j: (j * GRID + i, 0)),
        out_shape=jax.ShapeDtypeStruct((2 * N, DH), _f32),
    )(zv, zv, dv, w2, b2r)


def _final_body(za, zbr, dv, o):
    d = dv[...][:, :1]
    o[...] = jnp.concatenate([za[...] * d, zbr[...] * d], axis=1)


def _tc_final(zv, dv):
    return pl.pallas_call(
        _final_body,
        grid=(GRID,),
        in_specs=[pl.BlockSpec((BM, DH), lambda i: (i, 0)),
                  pl.BlockSpec((BM, DH), lambda i: (GRID + i, 0)),
                  pl.BlockSpec((BM, DEGW), lambda i: (i, 0))],
        out_specs=pl.BlockSpec((BM, D), lambda i: (i, 0)),
        out_shape=jax.ShapeDtypeStruct((N, D), _f32),
    )(zv, zv, dv)


def kernel(X, W1, b1, W2, b2, node_idx, edge_idx):
    b1r = b1.reshape(1, D)
    b2r = b2.reshape(1, D)
    cidx3 = jnp.concatenate([node_idx, edge_idx]).reshape(DROW3, DK, DCH)
    s_node = jnp.concatenate([node_idx, node_idx + N])
    s_edge = jnp.concatenate([edge_idx, edge_idx + N])
    d_node = node_idx.reshape(NROW3, K, CHUNK)
    d_edge = edge_idx.reshape(NROW3, K, CHUNK)
    deg = _sc_degrees(cidx3)
    dv, de, y1 = _tc_mm1_prep(X, W1, b1r, deg)
    ze = _sc_segsum_scaled(s_node, d_edge, y1, de)
    zv = _sc_segsum(s_edge, d_node, ze)
    y2 = _tc_mid(zv, dv, W2, b2r)
    z2 = _sc_segsum_scaled(s_node, d_edge, y2, de)
    return _sc_segsum_final(s_edge, d_node, z2, dv)


# speedup vs baseline: 7.3748x; 1.0013x over previous
import functools

import jax
import jax.numpy as jnp
from jax import lax
from jax.experimental import pallas as pl
from jax.experimental.pallas import tpu as pltpu
from jax.experimental.pallas import tpu_sc as plsc

N = 10000
NNZ = 160000
D = 256
DH = 128
NS = 16
PER_TILE = NNZ // NS
CHUNK = 40
K = 5
BODY = K * CHUNK
NBODY = PER_TILE // BODY
NROW3 = NNZ // BODY
NPAD = 10240
RPT = NPAD // NS
OCH = 80
DEGW = 16
DCH = 125
DK = 4
DNB = PER_TILE // (DK * DCH)
DROW3 = 2 * NNZ // (DK * DCH)
BM = 1000
GRID = N // BM

_f32 = jnp.float32
_mesh = plsc.VectorSubcoreMesh(core_axis_name="c", subcore_axis_name="s")


@functools.partial(
    pl.kernel,
    mesh=_mesh,
    out_type=jax.ShapeDtypeStruct((2 * N, DEGW), _f32),
    scratch_types=[pltpu.VMEM((DK, DCH), jnp.int32),
                   pltpu.VMEM((DK, DCH), jnp.int32),
                   pltpu.VMEM((DCH, DEGW), _f32),
                   pltpu.VMEM((OCH, DEGW), _f32),
                   pltpu.VMEM((OCH, DEGW), _f32),
                   pltpu.VMEM_SHARED((NPAD, DEGW), _f32),
                   pltpu.SemaphoreType.DMA,
                   pltpu.SemaphoreType.DMA,
                   pltpu.SemaphoreType.DMA],
)
def _sc_degrees(cidx3, deg_out, didxA, didxB, onesb, zb, ob, acc,
                ssem, isemA, isemB):
    c = lax.axis_index("c")
    s = lax.axis_index("s")
    ones16 = jnp.ones((16,), _f32)
    zero16 = jnp.zeros((16,), _f32)
    for i in range(DCH):
        onesb[i, :] = ones16
    for i in range(OCH):
        zb[i, :] = zero16

    zcs = [pltpu.async_copy(zb, acc.at[pl.ds(s * RPT + k * OCH, OCH)],
                            isemA) for k in range(RPT // OCH)]
    for zc in zcs:
        zc.wait()
    plsc.subcore_barrier()

    rb = c * (DROW3 // 2) + s * DNB
    pltpu.sync_copy(cidx3.at[rb], didxA)

    def body(j, carry):
        rA2 = rb + jnp.minimum(2 * j + 2, DNB - 1)
        sA = [pltpu.async_copy(onesb, acc.at[didxA.at[i]], ssem, add=True)
              for i in range(DK)]
        iB = pltpu.async_copy(cidx3.at[rb + 2 * j + 1], didxB, isemB)
        for i in range(DK):
            sA[i].wait()
        iA = pltpu.async_copy(cidx3.at[rA2], didxA, isemA)
        iB.wait()
        sB = [pltpu.async_copy(onesb, acc.at[didxB.at[i]], ssem, add=True)
              for i in range(DK)]
        for i in range(DK):
            sB[i].wait()
        iA.wait()
        return carry
    lax.fori_loop(0, DNB // 2, body, None)
    plsc.subcore_barrier()

    nch = jnp.minimum(jnp.maximum(N - s * RPT, 0), RPT) // OCH

    def obody(k, carry):
        r = s * RPT + k * OCH
        pltpu.sync_copy(acc.at[pl.ds(r, OCH)], ob)
        pltpu.sync_copy(ob, deg_out.at[pl.ds(c * N + r, OCH)])
        return carry
    lax.fori_loop(0, nch, obody, None)


def _seg_impl(src2, dst3, tab, sc16, out,
              sidxA, sidxB, didxA, didxB,
              rows0, rows1, rows2, rows3, rows4, stg, scb, acc,
              g0, g1, g2, g3, g4, ssem, isemA, isemB, scaled,
              concat_out=False):
    c = lax.axis_index("c")
    s = lax.axis_index("s")
    rows = (rows0, rows1, rows2, rows3, rows4)
    gsems = (g0, g1, g2, g3, g4)
    zero16 = jnp.zeros((16,), _f32)
    for i in range(OCH):
        for k in range(DH // 16):
            stg[i, pl.ds(k * 16, 16)] = zero16

    zcs = [pltpu.async_copy(stg, acc.at[pl.ds(s * RPT + k * OCH, OCH)],
                            isemA) for k in range(RPT // OCH)]
    for zc in zcs:
        zc.wait()
    plsc.subcore_barrier()

    sbase = c * NNZ + s * PER_TILE
    rbase = s * NBODY

    pltpu.sync_copy(src2.at[pl.ds(sbase, BODY)], sidxA)
    pltpu.sync_copy(dst3.at[rbase], didxA)

    def body(j, carry):
        jA, jB, jA2 = 2 * j, 2 * j + 1, 2 * j + 2
        jA2c = jnp.minimum(jA2, NBODY - 1)

        gA = [pltpu.async_copy(tab.at[sidxA.at[pl.ds(i * CHUNK, CHUNK)]],
                               rows[i], gsems[i]) for i in range(K)]
        iB0 = pltpu.async_copy(src2.at[pl.ds(sbase + jB * BODY, BODY)],
                               sidxB, isemB)
        iB1 = pltpu.async_copy(dst3.at[rbase + jB], didxB, isemB)
        sA = []
        for i in range(K):
            gA[i].wait()
            sA.append(pltpu.async_copy(rows[i], acc.at[didxA.at[i]], ssem,
                                       add=True))
        iA0 = pltpu.async_copy(src2.at[pl.ds(sbase + jA2c * BODY, BODY)],
                               sidxA, isemA)
        iB0.wait()
        iB1.wait()
        gB = []
        for i in range(K):
            sA[i].wait()
            gB.append(pltpu.async_copy(tab.at[sidxB.at[pl.ds(i * CHUNK,
                                                             CHUNK)]],
                                       rows[i], gsems[i]))
        iA1 = pltpu.async_copy(dst3.at[rbase + jA2c], didxA, isemA)
        sB = []
        for i in range(K):
            gB[i].wait()
            sB.append(pltpu.async_copy(rows[i], acc.at[didxB.at[i]], ssem,
                                       add=True))
        for i in range(K):
            sB[i].wait()
        iA0.wait()
        iA1.wait()
        return carry
    lax.fori_loop(0, NBODY // 2, body, None)
    plsc.subcore_barrier()

    nch = jnp.minimum(jnp.maximum(N - s * RPT, 0), RPT) // OCH

    def _oslice(r):
        if concat_out:
            return out.at[pl.ds(r, CHUNK), pl.ds(c * DH, DH)]
        return out.at[pl.ds(c * N + r, CHUNK)]

    def _oscale(buf, soff):
        for i in range(CHUNK):
            v = scb[soff + i, :]
            for k2 in range(DH // 16):
                buf[i, pl.ds(k2 * 16, 16)] = buf[i, pl.ds(k2 * 16, 16)] * v

    def obody(k, carry):
        r0 = s * RPT + (2 * k) * CHUNK
        r1 = r0 + CHUNK
        ia = pltpu.async_copy(acc.at[pl.ds(r0, CHUNK)], rows0, g0)
        ib = pltpu.async_copy(acc.at[pl.ds(r1, CHUNK)], rows1, g1)
        if scaled:
            sa = pltpu.async_copy(sc16.at[pl.ds(r0, CHUNK)],
                                  scb.at[pl.ds(0, CHUNK)], g2)
            sb = pltpu.async_copy(sc16.at[pl.ds(r1, CHUNK)],
                                  scb.at[pl.ds(CHUNK, CHUNK)], g3)
        ia.wait()
        if scaled:
            sa.wait()
            _oscale(rows0, 0)
        oa = pltpu.async_copy(rows0, _oslice(r0), g4)
        ib.wait()
        if scaled:
            sb.wait()
            _oscale(rows1, CHUNK)
        ob = pltpu.async_copy(rows1, _oslice(r1), ssem)
        oa.wait()
        ob.wait()
        return carry
    lax.fori_loop(0, nch * OCH // (2 * CHUNK), obody, None)


_SEG_SCRATCH = ([pltpu.VMEM((BODY,), jnp.int32),
                 pltpu.VMEM((BODY,), jnp.int32),
                 pltpu.VMEM((K, CHUNK), jnp.int32),
                 pltpu.VMEM((K, CHUNK), jnp.int32)]
                + [pltpu.VMEM((CHUNK, DH), _f32)] * 5
                + [pltpu.VMEM((OCH, DH), _f32)])
_SEG_SEMS = [pltpu.SemaphoreType.DMA] * 8


@functools.partial(
    pl.kernel,
    mesh=_mesh,
    out_type=jax.ShapeDtypeStruct((2 * N, DH), _f32),
    scratch_types=(_SEG_SCRATCH
                   + [pltpu.VMEM_SHARED((NPAD, DH), _f32)]
                   + _SEG_SEMS),
)
def _sc_segsum(src2, dst3, tab, out,
               sidxA, sidxB, didxA, didxB,
               rows0, rows1, rows2, rows3, rows4, stg, acc,
               g0, g1, g2, g3, g4, ssem, isemA, isemB):
    _seg_impl(src2, dst3, tab, None, out,
              sidxA, sidxB, didxA, didxB,
              rows0, rows1, rows2, rows3, rows4, stg, None, acc,
              g0, g1, g2, g3, g4, ssem, isemA, isemB, scaled=False)


@functools.partial(
    pl.kernel,
    mesh=_mesh,
    out_type=jax.ShapeDtypeStruct((2 * N, DH), _f32),
    scratch_types=(_SEG_SCRATCH
                   + [pltpu.VMEM((OCH, DEGW), _f32)]
                   + [pltpu.VMEM_SHARED((NPAD, DH), _f32)]
                   + _SEG_SEMS),
)
def _sc_segsum_scaled(src2, dst3, tab, sc16, out,
                      sidxA, sidxB, didxA, didxB,
                      rows0, rows1, rows2, rows3, rows4, stg, scb, acc,
                      g0, g1, g2, g3, g4, ssem, isemA, isemB):
    _seg_impl(src2, dst3, tab, sc16, out,
              sidxA, sidxB, didxA, didxB,
              rows0, rows1, rows2, rows3, rows4, stg, scb, acc,
              g0, g1, g2, g3, g4, ssem, isemA, isemB, scaled=True)


@functools.partial(
    pl.kernel,
    mesh=_mesh,
    out_type=jax.ShapeDtypeStruct((N, D), _f32),
    scratch_types=(_SEG_SCRATCH
                   + [pltpu.VMEM((OCH, DEGW), _f32)]
                   + [pltpu.VMEM_SHARED((NPAD, DH), _f32)]
                   + _SEG_SEMS),
)
def _sc_segsum_final(src2, dst3, tab, sc16, out,
                     sidxA, sidxB, didxA, didxB,
                     rows0, rows1, rows2, rows3, rows4, stg, scb, acc,
                     g0, g1, g2, g3, g4, ssem, isemA, isemB):
    _seg_impl(src2, dst3, tab, sc16, out,
              sidxA, sidxB, didxA, didxB,
              rows0, rows1, rows2, rows3, rows4, stg, scb, acc,
              g0, g1, g2, g3, g4, ssem, isemA, isemB, scaled=True,
              concat_out=True)


def _mm1p_body(x, w, b, dvd, ded, dv, de, y1):
    d = lax.rsqrt(dvd[...])
    dv[...] = d
    de[...] = 1.0 / ded[...]
    yy = lax.dot_general(x[...], w[...], (((1,), (1,)), ((), ())),
                         preferred_element_type=_f32)
    y1[...] = (yy + b[...]) * d[:, :1]


def _tc_mm1_prep(x, w1, b1r, deg):
    return pl.pallas_call(
        _mm1p_body,
        grid=(GRID, 2),
        in_specs=[pl.BlockSpec((BM, D), lambda i, j: (i, 0)),
                  pl.BlockSpec((DH, D), lambda i, j: (j, 0)),
                  pl.BlockSpec((1, DH), lambda i, j: (0, j)),
                  pl.BlockSpec((BM, DEGW), lambda i, j: (i, 0)),
                  pl.BlockSpec((BM, DEGW), lambda i, j: (GRID + i, 0))],
        out_specs=[pl.BlockSpec((BM, DEGW), lambda i, j: (i, 0)),
                   pl.BlockSpec((BM, DEGW), lambda i, j: (i, 0)),
                   pl.BlockSpec((BM, DH), lambda i, j: (j * GRID + i, 0))],
        out_shape=[jax.ShapeDtypeStruct((N, DEGW), _f32),
                   jax.ShapeDtypeStruct((N, DEGW), _f32),
                   jax.ShapeDtypeStruct((2 * N, DH), _f32)],
    )(x, w1, b1r, deg, deg)


def _mid_body(za, zbr, dv, w, b, y):
    d = dv[...][:, :1]
    h = jnp.concatenate([jnp.maximum(za[...] * d, 0.0),
                         jnp.maximum(zbr[...] * d, 0.0)], axis=1)
    yy = lax.dot_general(h, w[...], (((1,), (1,)), ((), ())),
                         preferred_element_type=_f32)
    y[...] = (yy + b[...]) * d


def _tc_mid(zv, dv, w2, b2r):
    return pl.pallas_call(
        _mid_body,
        grid=(GRID, 2),
        in_specs=[pl.BlockSpec((BM, DH), lambda i, j: (i, 0)),
                  pl.BlockSpec((BM, DH), lambda i, j: (GRID + i, 0)),
                  pl.BlockSpec((BM, DEGW), lambda i, j: (i, 0)),
                  pl.BlockSpec((DH, D), lambda i, j: (j, 0)),
                  pl.BlockSpec((1, DH), lambda i, j: (0, j))],
        out_specs=pl.BlockSpec((BM, DH), lambda i, j: (j * GRID + i, 0)),
        out_shape=jax.ShapeDtypeStruct((2 * N, DH), _f32),
    )(zv, zv, dv, w2, b2r)


def kernel(X, W1, b1, W2, b2, node_idx, edge_idx):
    b1r = b1.reshape(1, D)
    b2r = b2.reshape(1, D)
    cidx3 = jnp.concatenate([node_idx, edge_idx]).reshape(DROW3, DK, DCH)
    s_node = jnp.concatenate([node_idx, node_idx + N])
    s_edge = jnp.concatenate([edge_idx, edge_idx + N])
    d_node = node_idx.reshape(NROW3, K, CHUNK)
    d_edge = edge_idx.reshape(NROW3, K, CHUNK)
    deg = _sc_degrees(cidx3)
    dv, de, y1 = _tc_mm1_prep(X, W1, b1r, deg)
    ze = _sc_segsum_scaled(s_node, d_edge, y1, de)
    zv = _sc_segsum(s_edge, d_node, ze)
    y2 = _tc_mid(zv, dv, W2, b2r)
    z2 = _sc_segsum_scaled(s_node, d_edge, y2, de)
    return _sc_segsum_final(s_edge, d_node, z2, dv)


# speedup vs baseline: 7.4003x; 1.0035x over previous
import functools

import jax
import jax.numpy as jnp
from jax import lax
from jax.experimental import pallas as pl
from jax.experimental.pallas import tpu as pltpu
from jax.experimental.pallas import tpu_sc as plsc

N = 10000
NNZ = 160000
D = 256
DH = 128
NS = 16
PER_TILE = NNZ // NS
CHUNK = 40
K = 5
BODY = K * CHUNK
NBODY = PER_TILE // BODY
NROW3 = NNZ // BODY
NPAD = 10240
RPT = NPAD // NS
OCH = 80
DEGW = 16
DCH = 125
DK = 4
DNB = PER_TILE // (DK * DCH)
DROW3 = 2 * NNZ // (DK * DCH)
BM = 1000
GRID = N // BM

_f32 = jnp.float32
_mesh = plsc.VectorSubcoreMesh(core_axis_name="c", subcore_axis_name="s")


@functools.partial(
    pl.kernel,
    mesh=_mesh,
    out_type=jax.ShapeDtypeStruct((2 * N, DEGW), _f32),
    scratch_types=[pltpu.VMEM((DK, DCH), jnp.int32),
                   pltpu.VMEM((DK, DCH), jnp.int32),
                   pltpu.VMEM((DCH, DEGW), _f32),
                   pltpu.VMEM((OCH, DEGW), _f32),
                   pltpu.VMEM((OCH, DEGW), _f32),
                   pltpu.VMEM_SHARED((NPAD, DEGW), _f32),
                   pltpu.SemaphoreType.DMA,
                   pltpu.SemaphoreType.DMA,
                   pltpu.SemaphoreType.DMA],
)
def _sc_degrees(cidx3, deg_out, didxA, didxB, onesb, zb, ob, acc,
                ssem, isemA, isemB):
    c = lax.axis_index("c")
    s = lax.axis_index("s")
    ones16 = jnp.ones((16,), _f32)
    zero16 = jnp.zeros((16,), _f32)
    for i in range(DCH):
        onesb[i, :] = ones16
    for i in range(OCH):
        zb[i, :] = zero16

    zcs = [pltpu.async_copy(zb, acc.at[pl.ds(s * RPT + k * OCH, OCH)],
                            isemA) for k in range(RPT // OCH)]
    for zc in zcs:
        zc.wait()
    plsc.subcore_barrier()

    rb = c * (DROW3 // 2) + s * DNB
    pltpu.sync_copy(cidx3.at[rb], didxA)

    def body(j, carry):
        rA2 = rb + jnp.minimum(2 * j + 2, DNB - 1)
        sA = [pltpu.async_copy(onesb, acc.at[didxA.at[i]], ssem, add=True)
              for i in range(DK)]
        iB = pltpu.async_copy(cidx3.at[rb + 2 * j + 1], didxB, isemB)
        for i in range(DK):
            sA[i].wait()
        iA = pltpu.async_copy(cidx3.at[rA2], didxA, isemA)
        iB.wait()
        sB = [pltpu.async_copy(onesb, acc.at[didxB.at[i]], ssem, add=True)
              for i in range(DK)]
        for i in range(DK):
            sB[i].wait()
        iA.wait()
        return carry
    lax.fori_loop(0, DNB // 2, body, None)
    plsc.subcore_barrier()

    nch = jnp.minimum(jnp.maximum(N - s * RPT, 0), RPT) // OCH

    def obody(k, carry):
        r = s * RPT + k * OCH
        pltpu.sync_copy(acc.at[pl.ds(r, OCH)], ob)
        pltpu.sync_copy(ob, deg_out.at[pl.ds(c * N + r, OCH)])
        return carry
    lax.fori_loop(0, nch, obody, None)


def _seg_impl(src2, dst3, tab, sc16, out,
              sidxA, sidxB, didxA, didxB,
              rows0, rows1, rows2, rows3, rows4, stg, scb, acc,
              g0, g1, g2, g3, g4, ssem, ssemB, isemA, isemB, scaled,
              concat_out=False):
    c = lax.axis_index("c")
    s = lax.axis_index("s")
    rows = (rows0, rows1, rows2, rows3, rows4)
    gsems = (g0, g1, g2, g3, g4)
    zero16 = jnp.zeros((16,), _f32)
    for i in range(OCH):
        for k in range(DH // 16):
            stg[i, pl.ds(k * 16, 16)] = zero16

    zcs = [pltpu.async_copy(stg, acc.at[pl.ds(s * RPT + k * OCH, OCH)],
                            isemA) for k in range(RPT // OCH)]
    for zc in zcs:
        zc.wait()
    plsc.subcore_barrier()

    sbase = c * NNZ + s * PER_TILE
    rbase = s * NBODY

    pltpu.sync_copy(src2.at[pl.ds(sbase, BODY)], sidxA)
    pltpu.sync_copy(dst3.at[rbase], didxA)

    def body(j, carry):
        jA, jB, jA2 = 2 * j, 2 * j + 1, 2 * j + 2
        jA2c = jnp.minimum(jA2, NBODY - 1)

        @pl.when(j > 0)
        def _():
            for i in range(K):
                pltpu.make_async_copy(tab.at[pl.ds(0, CHUNK)], rows[i],
                                      ssemB).wait()

        gA = [pltpu.async_copy(tab.at[sidxA.at[pl.ds(i * CHUNK, CHUNK)]],
                               rows[i], gsems[i]) for i in range(K)]
        iB0 = pltpu.async_copy(src2.at[pl.ds(sbase + jB * BODY, BODY)],
                               sidxB, isemB)
        iB1 = pltpu.async_copy(dst3.at[rbase + jB], didxB, isemB)
        sA = []
        for i in range(K):
            gA[i].wait()
            sA.append(pltpu.async_copy(rows[i], acc.at[didxA.at[i]], ssem,
                                       add=True))
        iA0 = pltpu.async_copy(src2.at[pl.ds(sbase + jA2c * BODY, BODY)],
                               sidxA, isemA)
        iB0.wait()
        iB1.wait()
        gB = []
        for i in range(K):
            sA[i].wait()
            gB.append(pltpu.async_copy(tab.at[sidxB.at[pl.ds(i * CHUNK,
                                                             CHUNK)]],
                                       rows[i], gsems[i]))
        iA1 = pltpu.async_copy(dst3.at[rbase + jA2c], didxA, isemA)
        for i in range(K):
            gB[i].wait()
            pltpu.async_copy(rows[i], acc.at[didxB.at[i]], ssemB, add=True)
        iA0.wait()
        iA1.wait()
        return carry
    lax.fori_loop(0, NBODY // 2, body, None)
    for i in range(K):
        pltpu.make_async_copy(tab.at[pl.ds(0, CHUNK)], rows[i], ssemB).wait()
    plsc.subcore_barrier()

    nch = jnp.minimum(jnp.maximum(N - s * RPT, 0), RPT) // OCH

    def _oslice(r):
        if concat_out:
            return out.at[pl.ds(r, CHUNK), pl.ds(c * DH, DH)]
        return out.at[pl.ds(c * N + r, CHUNK)]

    def _oscale(buf, soff):
        for i in range(CHUNK):
            v = scb[soff + i, :]
            for k2 in range(DH // 16):
                buf[i, pl.ds(k2 * 16, 16)] = buf[i, pl.ds(k2 * 16, 16)] * v

    def obody(k, carry):
        r0 = s * RPT + (2 * k) * CHUNK
        r1 = r0 + CHUNK
        ia = pltpu.async_copy(acc.at[pl.ds(r0, CHUNK)], rows0, g0)
        ib = pltpu.async_copy(acc.at[pl.ds(r1, CHUNK)], rows1, g1)
        if scaled:
            sa = pltpu.async_copy(sc16.at[pl.ds(r0, CHUNK)],
                                  scb.at[pl.ds(0, CHUNK)], g2)
            sb = pltpu.async_copy(sc16.at[pl.ds(r1, CHUNK)],
                                  scb.at[pl.ds(CHUNK, CHUNK)], g3)
        ia.wait()
        if scaled:
            sa.wait()
            _oscale(rows0, 0)
        oa = pltpu.async_copy(rows0, _oslice(r0), g4)
        ib.wait()
        if scaled:
            sb.wait()
            _oscale(rows1, CHUNK)
        ob = pltpu.async_copy(rows1, _oslice(r1), ssem)
        oa.wait()
        ob.wait()
        return carry
    lax.fori_loop(0, nch * OCH // (2 * CHUNK), obody, None)


_SEG_SCRATCH = ([pltpu.VMEM((BODY,), jnp.int32),
                 pltpu.VMEM((BODY,), jnp.int32),
                 pltpu.VMEM((K, CHUNK), jnp.int32),
                 pltpu.VMEM((K, CHUNK), jnp.int32)]
                + [pltpu.VMEM((CHUNK, DH), _f32)] * 5
                + [pltpu.VMEM((OCH, DH), _f32)])
_SEG_SEMS = [pltpu.SemaphoreType.DMA] * 9


@functools.partial(
    pl.kernel,
    mesh=_mesh,
    out_type=jax.ShapeDtypeStruct((2 * N, DH), _f32),
    scratch_types=(_SEG_SCRATCH
                   + [pltpu.VMEM_SHARED((NPAD, DH), _f32)]
                   + _SEG_SEMS),
)
def _sc_segsum(src2, dst3, tab, out,
               sidxA, sidxB, didxA, didxB,
               rows0, rows1, rows2, rows3, rows4, stg, acc,
               g0, g1, g2, g3, g4, ssem, ssemB, isemA, isemB):
    _seg_impl(src2, dst3, tab, None, out,
              sidxA, sidxB, didxA, didxB,
              rows0, rows1, rows2, rows3, rows4, stg, None, acc,
              g0, g1, g2, g3, g4, ssem, ssemB, isemA, isemB, scaled=False)


@functools.partial(
    pl.kernel,
    mesh=_mesh,
    out_type=jax.ShapeDtypeStruct((2 * N, DH), _f32),
    scratch_types=(_SEG_SCRATCH
                   + [pltpu.VMEM((OCH, DEGW), _f32)]
                   + [pltpu.VMEM_SHARED((NPAD, DH), _f32)]
                   + _SEG_SEMS),
)
def _sc_segsum_scaled(src2, dst3, tab, sc16, out,
                      sidxA, sidxB, didxA, didxB,
                      rows0, rows1, rows2, rows3, rows4, stg, scb, acc,
                      g0, g1, g2, g3, g4, ssem, ssemB, isemA, isemB):
    _seg_impl(src2, dst3, tab, sc16, out,
              sidxA, sidxB, didxA, didxB,
              rows0, rows1, rows2, rows3, rows4, stg, scb, acc,
              g0, g1, g2, g3, g4, ssem, ssemB, isemA, isemB, scaled=True)


@functools.partial(
    pl.kernel,
    mesh=_mesh,
    out_type=jax.ShapeDtypeStruct((N, D), _f32),
    scratch_types=(_SEG_SCRATCH
                   + [pltpu.VMEM((OCH, DEGW), _f32)]
                   + [pltpu.VMEM_SHARED((NPAD, DH), _f32)]
                   + _SEG_SEMS),
)
def _sc_segsum_final(src2, dst3, tab, sc16, out,
                     sidxA, sidxB, didxA, didxB,
                     rows0, rows1, rows2, rows3, rows4, stg, scb, acc,
                     g0, g1, g2, g3, g4, ssem, ssemB, isemA, isemB):
    _seg_impl(src2, dst3, tab, sc16, out,
              sidxA, sidxB, didxA, didxB,
              rows0, rows1, rows2, rows3, rows4, stg, scb, acc,
              g0, g1, g2, g3, g4, ssem, ssemB, isemA, isemB, scaled=True,
              concat_out=True)


def _mm1p_body(x, w, b, dvd, ded, dv, de, y1):
    d = lax.rsqrt(dvd[...])
    dv[...] = d
    de[...] = 1.0 / ded[...]
    yy = lax.dot_general(x[...], w[...], (((1,), (1,)), ((), ())),
                         preferred_element_type=_f32)
    y1[...] = (yy + b[...]) * d[:, :1]


def _tc_mm1_prep(x, w1, b1r, deg):
    return pl.pallas_call(
        _mm1p_body,
        grid=(GRID, 2),
        in_specs=[pl.BlockSpec((BM, D), lambda i, j: (i, 0)),
                  pl.BlockSpec((DH, D), lambda i, j: (j, 0)),
                  pl.BlockSpec((1, DH), lambda i, j: (0, j)),
                  pl.BlockSpec((BM, DEGW), lambda i, j: (i, 0)),
                  pl.BlockSpec((BM, DEGW), lambda i, j: (GRID + i, 0))],
        out_specs=[pl.BlockSpec((BM, DEGW), lambda i, j: (i, 0)),
                   pl.BlockSpec((BM, DEGW), lambda i, j: (i, 0)),
                   pl.BlockSpec((BM, DH), lambda i, j: (j * GRID + i, 0))],
        out_shape=[jax.ShapeDtypeStruct((N, DEGW), _f32),
                   jax.ShapeDtypeStruct((N, DEGW), _f32),
                   jax.ShapeDtypeStruct((2 * N, DH), _f32)],
    )(x, w1, b1r, deg, deg)


def _mid_body(za, zbr, dv, w, b, y):
    d = dv[...][:, :1]
    h = jnp.concatenate([jnp.maximum(za[...] * d, 0.0),
                         jnp.maximum(zbr[...] * d, 0.0)], axis=1)
    yy = lax.dot_general(h, w[...], (((1,), (1,)), ((), ())),
                         preferred_element_type=_f32)
    y[...] = (yy + b[...]) * d


def _tc_mid(zv, dv, w2, b2r):
    return pl.pallas_call(
        _mid_body,
        grid=(GRID, 2),
        in_specs=[pl.BlockSpec((BM, DH), lambda i, j: (i, 0)),
                  pl.BlockSpec((BM, DH), lambda i, j: (GRID + i, 0)),
                  pl.BlockSpec((BM, DEGW), lambda i, j: (i, 0)),
                  pl.BlockSpec((DH, D), lambda i, j: (j, 0)),
                  pl.BlockSpec((1, DH), lambda i, j: (0, j))],
        out_specs=pl.BlockSpec((BM, DH), lambda i, j: (j * GRID + i, 0)),
        out_shape=jax.ShapeDtypeStruct((2 * N, DH), _f32),
    )(zv, zv, dv, w2, b2r)


def kernel(X, W1, b1, W2, b2, node_idx, edge_idx):
    b1r = b1.reshape(1, D)
    b2r = b2.reshape(1, D)
    cidx3 = jnp.concatenate([node_idx, edge_idx]).reshape(DROW3, DK, DCH)
    s_node = jnp.concatenate([node_idx, node_idx + N])
    s_edge = jnp.concatenate([edge_idx, edge_idx + N])
    d_node = node_idx.reshape(NROW3, K, CHUNK)
    d_edge = edge_idx.reshape(NROW3, K, CHUNK)
    deg = _sc_degrees(cidx3)
    dv, de, y1 = _tc_mm1_prep(X, W1, b1r, deg)
    ze = _sc_segsum_scaled(s_node, d_edge, y1, de)
    zv = _sc_segsum(s_edge, d_node, ze)
    y2 = _tc_mid(zv, dv, W2, b2r)
    z2 = _sc_segsum_scaled(s_node, d_edge, y2, de)
    return _sc_segsum_final(s_edge, d_node, z2, dv)
